# baseline (speedup 1.0000x reference)
import functools
from math import sqrt

import numpy as np
import jax
import jax.numpy as jnp
from jax import lax
from jax.experimental import pallas as pl
from jax.experimental.pallas import tpu as pltpu

F32 = jnp.float32
_PREC = lax.Precision.HIGHEST
BF16 = jnp.bfloat16

_CONSTS = {}


def _threefry2x32(k0, k1, c0, c1):
    k0, k1 = np.uint32(k0), np.uint32(k1)
    x0 = (c0 + k0).astype(np.uint32)
    x1 = (c1 + k1).astype(np.uint32)
    ks = [k0, k1, np.uint32(np.uint32(k0) ^ np.uint32(k1) ^ np.uint32(0x1BD11BDA))]
    rots = [[13, 15, 26, 6], [17, 29, 16, 24]]
    for g in range(5):
        for r in rots[g % 2]:
            x0 = (x0 + x1).astype(np.uint32)
            x1 = ((x1 << np.uint32(r)) | (x1 >> np.uint32(32 - r))).astype(np.uint32) ^ x0
        x0 = (x0 + ks[(g + 1) % 3]).astype(np.uint32)
        x1 = (x1 + ks[(g + 2) % 3] + np.uint32(g + 1)).astype(np.uint32)
    return x0, x1


def _np_randint(shape, span):
    o0, o1 = _threefry2x32(np.uint32(0), np.uint32(42),
                           np.zeros(2, np.uint32), np.arange(2, dtype=np.uint32))
    n = int(np.prod(shape))

    def bits(sk0, sk1):
        c = np.arange(n, dtype=np.uint64)
        hi = (c >> np.uint64(32)).astype(np.uint32)
        lo = (c & np.uint64(0xFFFFFFFF)).astype(np.uint32)
        x0, x1 = _threefry2x32(sk0, sk1, hi, lo)
        return (x0 ^ x1).astype(np.uint32)

    u = bits(o0[0], o1[0])
    v = bits(o0[1], o1[1])
    be = np.uint32(span)
    bh = np.uint32((np.uint64(65536 % span) ** 2) % np.uint64(span))
    out = ((u % be) * bh + (v % be)) % be
    return out.astype(np.int32).reshape(shape)


def _sample_counts_t(l_q: int, l_k: int):
    ck = (l_q, l_k)
    if ck not in _CONSTS:
        u_part = min(int(3 * np.ceil(np.log(l_k))), l_k)
        idx_np = _np_randint((l_q, u_part), l_k)
        cnt = np.zeros((l_k, l_q), dtype=np.float32)
        rows = np.broadcast_to(np.arange(l_q)[:, None], idx_np.shape)
        np.add.at(cnt, (idx_np, rows), 1.0)
        neg = np.where(cnt > 0.0, 0.0, -1e30).astype(np.float32)
        _CONSTS[ck] = (jnp.asarray(cnt), jnp.asarray(neg))
    return _CONSTS[ck]


def _proj_body(x_ref, w_ref, b_ref, o_ref):
    o_ref[...] = (
        lax.dot_general(
            x_ref[...], w_ref[...], (((1,), (1,)), ((), ())),
            precision=_PREC, preferred_element_type=F32,
        )
        + b_ref[...]
    )


def _proj_split_body(x_ref, w_ref, b_ref, hi_ref, lo_ref):
    y = (
        lax.dot_general(
            x_ref[...], w_ref[...], (((1,), (1,)), ((), ())),
            precision=_PREC, preferred_element_type=F32,
        )
        + b_ref[...]
    )
    hi = y.astype(BF16)
    hi_ref[...] = hi
    lo_ref[...] = (y - hi.astype(F32)).astype(BF16)


def _project(x2d, w, b, tile, split=False):
    n, d_in = x2d.shape
    d_out = w.shape[0]
    in_specs = [
        pl.BlockSpec((tile, d_in), lambda i: (i, 0)),
        pl.BlockSpec((d_out, d_in), lambda i: (0, 0)),
        pl.BlockSpec((1, d_out), lambda i: (0, 0)),
    ]
    if not split:
        return pl.pallas_call(
            _proj_body,
            grid=(n // tile,),
            in_specs=in_specs,
            out_specs=pl.BlockSpec((tile, d_out), lambda i: (i, 0)),
            out_shape=jax.ShapeDtypeStruct((n, d_out), F32),
        )(x2d, w, b.reshape(1, d_out))
    return pl.pallas_call(
        _proj_split_body,
        grid=(n // tile,),
        in_specs=in_specs,
        out_specs=[
            pl.BlockSpec((tile, d_out), lambda i: (i, 0)),
            pl.BlockSpec((tile, d_out), lambda i: (i, 0)),
        ],
        out_shape=[
            jax.ShapeDtypeStruct((n, d_out), BF16),
            jax.ShapeDtypeStruct((n, d_out), BF16),
        ],
    )(x2d, w, b.reshape(1, d_out))


def _m_body(qh_ref, ql_ref, kh_ref, kl_ref, c_ref, n_ref, m_ref, *, t_rows, l_k):
    t = pl.program_id(1)
    dims = (((1,), (1,)), ((), ()))
    qh = qh_ref[0, 0]
    ql = ql_ref[0, 0]
    kh = kh_ref[0, 0]
    kl = kl_ref[0, 0]
    st = (
        lax.dot_general(kh, qh, dims, preferred_element_type=F32)
        + lax.dot_general(kh, ql, dims, preferred_element_type=F32)
        + lax.dot_general(kl, qh, dims, preferred_element_type=F32)
    )
    c = c_ref[:, pl.ds(t * t_rows, t_rows)]
    neg = n_ref[:, pl.ds(t * t_rows, t_rows)]
    smax = jnp.max(st + neg, axis=0, keepdims=True)
    ssum = jnp.sum(st * c, axis=0, keepdims=True)
    m_ref[0, :, pl.ds(t * t_rows, t_rows)] = smax - ssum / l_k


def _topk_body(m_ref, oh_ref, idx_ref, *, u, l_q):
    m = m_ref[0]
    iota_r = lax.broadcasted_iota(jnp.int32, (1, l_q), 1)

    def body(j, mcur):
        mx = jnp.max(mcur)
        amax = jnp.min(jnp.where(mcur == mx, iota_r, l_q))
        oh_ref[0, pl.ds(j, 1), :] = (iota_r == amax).astype(F32)
        idx_ref[0, pl.ds(j, 1), :] = amax.astype(jnp.int32).reshape(1, 1)
        return jnp.where(iota_r == amax, -1e30, mcur)

    lax.fori_loop(0, u, body, m)


def _softmax_rows_body(oh_ref, qh_ref, ql_ref, kh_ref, kl_ref, p_ref, *, scale):
    sel = (((1,), (0,)), ((), ()))
    dims = (((1,), (1,)), ((), ()))
    oh16 = oh_ref[0].astype(BF16)
    qred = (
        lax.dot_general(oh16, qh_ref[0, 0], sel, preferred_element_type=F32)
        + lax.dot_general(oh16, ql_ref[0, 0], sel, preferred_element_type=F32)
    )
    qrh = qred.astype(BF16)
    qrl = (qred - qrh.astype(F32)).astype(BF16)
    kh = kh_ref[0, 0]
    kl = kl_ref[0, 0]
    s = (
        lax.dot_general(qrh, kh, dims, preferred_element_type=F32)
        + lax.dot_general(qrh, kl, dims, preferred_element_type=F32)
        + lax.dot_general(qrl, kh, dims, preferred_element_type=F32)
    ) * scale
    p = jnp.exp(s - jnp.max(s, axis=1, keepdims=True))
    p_ref[0] = p / jnp.sum(p, axis=1, keepdims=True)


def _merge_body(oh_ref, p_ref, v_ref, pcomb_ref, canon_ref, orows_ref,
                *, n_head, u, l_k):
    hu = n_head * u
    pall = p_ref[0]
    ohb = oh_ref[0]
    eq = lax.dot_general(ohb, ohb, (((1,), (1,)), ((), ())),
                         precision=_PREC, preferred_element_type=F32)
    cnt = jnp.sum(eq, axis=1, keepdims=True)
    ii = lax.broadcasted_iota(jnp.int32, (hu, hu), 0)
    jj = lax.broadcasted_iota(jnp.int32, (hu, hu), 1)
    prior = jnp.sum(eq * (jj < ii).astype(F32), axis=1, keepdims=True)
    canon = (prior == 0.0).astype(F32)
    base = (n_head - cnt) / (n_head * l_k)
    pc = base + lax.dot_general(eq, pall, (((1,), (0,)), ((), ())),
                                precision=_PREC, preferred_element_type=F32) / n_head
    pcomb_ref[0] = pc
    canon_ref[0] = canon
    orows_ref[0] = lax.dot_general(pc * canon, v_ref[0], (((1,), (0,)), ((), ())),
                                   precision=_PREC, preferred_element_type=F32)


def _attn_body(oh_ref, pc_ref, canon_ref, attn_ref, *, n_head, t_rows, l_k):
    t = pl.program_id(1)
    ohd = oh_ref[0, :, pl.ds(t * t_rows, t_rows)] * canon_ref[0]
    pd = pc_ref[0] * canon_ref[0]
    content = lax.dot_general(ohd, pd, (((0,), (0,)), ((), ())),
                              precision=_PREC, preferred_element_type=F32)
    selrow = lax.dot_general(ohd, jnp.ones((ohd.shape[0], 1), F32),
                             (((0,), (0,)), ((), ())),
                             precision=_PREC, preferred_element_type=F32)
    attn_ref[0] = content + (1.0 - selrow) * (1.0 / l_k)


def _out_body(oh_ref, canon_ref, orows_ref, v_ref, wo_ref, bo_ref, out_ref,
              *, n_head, t_rows, l_k):
    t = pl.program_id(1)
    ohd = oh_ref[0, :, pl.ds(t * t_rows, t_rows)] * canon_ref[0]
    meanv = jnp.sum(v_ref[0], axis=0, keepdims=True) / l_k
    selrow = lax.dot_general(ohd, jnp.ones((ohd.shape[0], 1), F32),
                             (((0,), (0,)), ((), ())),
                             precision=_PREC, preferred_element_type=F32)
    opre = lax.dot_general(ohd, orows_ref[0], (((0,), (0,)), ((), ())),
                           precision=_PREC, preferred_element_type=F32)
    opre = opre + (1.0 - selrow) * meanv
    y = lax.dot_general(opre, wo_ref[...], (((1,), (1,)), ((), ())),
                        precision=_PREC, preferred_element_type=F32) + bo_ref[...]
    out_ref[0] = y


def kernel(q, k, v, mask, Wq, bq, Wk, bk, Wv, bv, Wo, bo):
    del mask
    b_sz, l_q, d_model = q.shape
    l_k = k.shape[1]
    n_head = 12
    dk = d_model // n_head
    dv = Wv.shape[0]
    u = min(int(3 * np.ceil(np.log(l_q))), l_q)
    hu = n_head * u
    scale = 1.0 / sqrt(dk)
    t_rows = 256

    cnts_t, cneg_t = _sample_counts_t(l_q, l_k)

    qp_hi, qp_lo = _project(q.reshape(b_sz * l_q, d_model), Wq, bq, 512,
                            split=True)
    kp_hi, kp_lo = _project(k.reshape(b_sz * l_k, d_model), Wk, bk, 512,
                            split=True)
    vp = _project(v.reshape(b_sz * l_k, d_model), Wv, bv, 512)
    tr = lambda a, l: a.reshape(b_sz, l, n_head, dk).transpose(0, 2, 1, 3)
    q4h = tr(qp_hi, l_q)
    q4l = tr(qp_lo, l_q)
    k4h = tr(kp_hi, l_k)
    k4l = tr(kp_lo, l_k)
    v3 = vp.reshape(b_sz, l_k, dv)

    m = pl.pallas_call(
        functools.partial(_m_body, t_rows=t_rows, l_k=l_k),
        grid=(b_sz * n_head, l_q // t_rows),
        in_specs=[
            pl.BlockSpec((1, 1, t_rows, dk),
                         lambda bh, t, H=n_head: (bh // H, bh % H, t, 0)),
            pl.BlockSpec((1, 1, t_rows, dk),
                         lambda bh, t, H=n_head: (bh // H, bh % H, t, 0)),
            pl.BlockSpec((1, 1, l_k, dk),
                         lambda bh, t, H=n_head: (bh // H, bh % H, 0, 0)),
            pl.BlockSpec((1, 1, l_k, dk),
                         lambda bh, t, H=n_head: (bh // H, bh % H, 0, 0)),
            pl.BlockSpec((l_k, l_q), lambda bh, t: (0, 0)),
            pl.BlockSpec((l_k, l_q), lambda bh, t: (0, 0)),
        ],
        out_specs=pl.BlockSpec((1, 1, l_q), lambda bh, t: (bh, 0, 0)),
        out_shape=jax.ShapeDtypeStruct((b_sz * n_head, 1, l_q), F32),
    )(q4h, q4l, k4h, k4l, cnts_t, cneg_t)

    oh, top_idx = pl.pallas_call(
        functools.partial(_topk_body, u=u, l_q=l_q),
        grid=(b_sz * n_head,),
        in_specs=[pl.BlockSpec((1, 1, l_q), lambda bh: (bh, 0, 0))],
        out_specs=[
            pl.BlockSpec((1, u, l_q), lambda bh: (bh, 0, 0)),
            pl.BlockSpec((1, u, 1), lambda bh: (bh, 0, 0)),
        ],
        out_shape=[
            jax.ShapeDtypeStruct((b_sz * n_head, u, l_q), F32),
            jax.ShapeDtypeStruct((b_sz * n_head, u, 1), jnp.int32),
        ],
    )(m)

    pall = pl.pallas_call(
        functools.partial(_softmax_rows_body, scale=scale),
        grid=(b_sz * n_head,),
        in_specs=[
            pl.BlockSpec((1, u, l_q), lambda bh: (bh, 0, 0)),
            pl.BlockSpec((1, 1, l_q, dk),
                         lambda bh, H=n_head: (bh // H, bh % H, 0, 0)),
            pl.BlockSpec((1, 1, l_q, dk),
                         lambda bh, H=n_head: (bh // H, bh % H, 0, 0)),
            pl.BlockSpec((1, 1, l_k, dk),
                         lambda bh, H=n_head: (bh // H, bh % H, 0, 0)),
            pl.BlockSpec((1, 1, l_k, dk),
                         lambda bh, H=n_head: (bh // H, bh % H, 0, 0)),
        ],
        out_specs=pl.BlockSpec((1, u, l_k), lambda bh: (bh, 0, 0)),
        out_shape=jax.ShapeDtypeStruct((b_sz * n_head, u, l_k), F32),
    )(oh, q4h, q4l, k4h, k4l)

    ohb3 = oh.reshape(b_sz, hu, l_q)
    pall3 = pall.reshape(b_sz, hu, l_k)

    pcomb, canon, orows = pl.pallas_call(
        functools.partial(_merge_body, n_head=n_head, u=u, l_k=l_k),
        grid=(b_sz,),
        in_specs=[
            pl.BlockSpec((1, hu, l_q), lambda b: (b, 0, 0)),
            pl.BlockSpec((1, hu, l_k), lambda b: (b, 0, 0)),
            pl.BlockSpec((1, l_k, dv), lambda b: (b, 0, 0)),
        ],
        out_specs=[
            pl.BlockSpec((1, hu, l_k), lambda b: (b, 0, 0)),
            pl.BlockSpec((1, hu, 1), lambda b: (b, 0, 0)),
            pl.BlockSpec((1, hu, dv), lambda b: (b, 0, 0)),
        ],
        out_shape=[
            jax.ShapeDtypeStruct((b_sz, hu, l_k), F32),
            jax.ShapeDtypeStruct((b_sz, hu, 1), F32),
            jax.ShapeDtypeStruct((b_sz, hu, dv), F32),
        ],
    )(ohb3, pall3, v3)

    attn = pl.pallas_call(
        functools.partial(_attn_body, n_head=n_head, t_rows=t_rows, l_k=l_k),
        grid=(b_sz, l_q // t_rows),
        in_specs=[
            pl.BlockSpec((1, hu, l_q), lambda b, t: (b, 0, 0)),
            pl.BlockSpec((1, hu, l_k), lambda b, t: (b, 0, 0)),
            pl.BlockSpec((1, hu, 1), lambda b, t: (b, 0, 0)),
        ],
        out_specs=pl.BlockSpec((1, t_rows, l_k), lambda b, t: (b, t, 0)),
        out_shape=jax.ShapeDtypeStruct((b_sz, l_q, l_k), F32),
    )(ohb3, pcomb, canon)

    out = pl.pallas_call(
        functools.partial(_out_body, n_head=n_head, t_rows=t_rows, l_k=l_k),
        grid=(b_sz, l_q // t_rows),
        in_specs=[
            pl.BlockSpec((1, hu, l_q), lambda b, t: (b, 0, 0)),
            pl.BlockSpec((1, hu, 1), lambda b, t: (b, 0, 0)),
            pl.BlockSpec((1, hu, dv), lambda b, t: (b, 0, 0)),
            pl.BlockSpec((1, l_k, dv), lambda b, t: (b, 0, 0)),
            pl.BlockSpec((d_model, dv), lambda b, t: (0, 0)),
            pl.BlockSpec((1, d_model), lambda b, t: (0, 0)),
        ],
        out_specs=pl.BlockSpec((1, t_rows, d_model), lambda b, t: (b, t, 0)),
        out_shape=jax.ShapeDtypeStruct((b_sz, l_q, d_model), F32),
    )(ohb3, canon, orows, v3, Wo, bo.reshape(1, d_model))

    return (out, attn)


# speedup vs baseline: 1.0162x; 1.0162x over previous
import functools
from math import sqrt

import numpy as np
import jax
import jax.numpy as jnp
from jax import lax
from jax.experimental import pallas as pl
from jax.experimental.pallas import tpu as pltpu

F32 = jnp.float32
_PREC = lax.Precision.HIGHEST
BF16 = jnp.bfloat16

_CONSTS = {}


def _threefry2x32(k0, k1, c0, c1):
    k0, k1 = np.uint32(k0), np.uint32(k1)
    x0 = (c0 + k0).astype(np.uint32)
    x1 = (c1 + k1).astype(np.uint32)
    ks = [k0, k1, np.uint32(np.uint32(k0) ^ np.uint32(k1) ^ np.uint32(0x1BD11BDA))]
    rots = [[13, 15, 26, 6], [17, 29, 16, 24]]
    for g in range(5):
        for r in rots[g % 2]:
            x0 = (x0 + x1).astype(np.uint32)
            x1 = ((x1 << np.uint32(r)) | (x1 >> np.uint32(32 - r))).astype(np.uint32) ^ x0
        x0 = (x0 + ks[(g + 1) % 3]).astype(np.uint32)
        x1 = (x1 + ks[(g + 2) % 3] + np.uint32(g + 1)).astype(np.uint32)
    return x0, x1


def _np_randint(shape, span):
    o0, o1 = _threefry2x32(np.uint32(0), np.uint32(42),
                           np.zeros(2, np.uint32), np.arange(2, dtype=np.uint32))
    n = int(np.prod(shape))

    def bits(sk0, sk1):
        c = np.arange(n, dtype=np.uint64)
        hi = (c >> np.uint64(32)).astype(np.uint32)
        lo = (c & np.uint64(0xFFFFFFFF)).astype(np.uint32)
        x0, x1 = _threefry2x32(sk0, sk1, hi, lo)
        return (x0 ^ x1).astype(np.uint32)

    u = bits(o0[0], o1[0])
    v = bits(o0[1], o1[1])
    be = np.uint32(span)
    bh = np.uint32((np.uint64(65536 % span) ** 2) % np.uint64(span))
    out = ((u % be) * bh + (v % be)) % be
    return out.astype(np.int32).reshape(shape)


def _sample_counts_t(l_q: int, l_k: int):
    ck = (l_q, l_k)
    if ck not in _CONSTS:
        u_part = min(int(3 * np.ceil(np.log(l_k))), l_k)
        idx_np = _np_randint((l_q, u_part), l_k)
        cnt = np.zeros((l_k, l_q), dtype=np.float32)
        rows = np.broadcast_to(np.arange(l_q)[:, None], idx_np.shape)
        np.add.at(cnt, (idx_np, rows), 1.0)
        neg = np.where(cnt > 0.0, 0.0, -1e30).astype(np.float32)
        _CONSTS[ck] = (jnp.asarray(cnt), jnp.asarray(neg))
    return _CONSTS[ck]


def _proj_body(x_ref, w_ref, b_ref, o_ref):
    o_ref[...] = (
        lax.dot_general(
            x_ref[...], w_ref[...], (((1,), (1,)), ((), ())),
            precision=_PREC, preferred_element_type=F32,
        )
        + b_ref[...]
    )


def _proj_split_body(x_ref, w_ref, b_ref, hi_ref, lo_ref):
    y = (
        lax.dot_general(
            x_ref[...], w_ref[...], (((1,), (1,)), ((), ())),
            precision=_PREC, preferred_element_type=F32,
        )
        + b_ref[...]
    )
    hi = y.astype(BF16)
    hi_ref[...] = hi
    lo_ref[...] = (y - hi.astype(F32)).astype(BF16)


def _project(x2d, w, b, tile, split=False):
    n, d_in = x2d.shape
    d_out = w.shape[0]
    in_specs = [
        pl.BlockSpec((tile, d_in), lambda i: (i, 0)),
        pl.BlockSpec((d_out, d_in), lambda i: (0, 0)),
        pl.BlockSpec((1, d_out), lambda i: (0, 0)),
    ]
    if not split:
        return pl.pallas_call(
            _proj_body,
            grid=(n // tile,),
            in_specs=in_specs,
            out_specs=pl.BlockSpec((tile, d_out), lambda i: (i, 0)),
            out_shape=jax.ShapeDtypeStruct((n, d_out), F32),
        )(x2d, w, b.reshape(1, d_out))
    return pl.pallas_call(
        _proj_split_body,
        grid=(n // tile,),
        in_specs=in_specs,
        out_specs=[
            pl.BlockSpec((tile, d_out), lambda i: (i, 0)),
            pl.BlockSpec((tile, d_out), lambda i: (i, 0)),
        ],
        out_shape=[
            jax.ShapeDtypeStruct((n, d_out), BF16),
            jax.ShapeDtypeStruct((n, d_out), BF16),
        ],
    )(x2d, w, b.reshape(1, d_out))


def _m_body(qh_ref, ql_ref, kh_ref, kl_ref, c_ref, m_ref, *, t_rows, l_k):
    t = pl.program_id(1)
    dims = (((1,), (1,)), ((), ()))
    qh = qh_ref[0, 0]
    ql = ql_ref[0, 0]
    kh = kh_ref[0, 0]
    kl = kl_ref[0, 0]
    st = (
        lax.dot_general(kh, qh, dims, preferred_element_type=F32)
        + lax.dot_general(kh, ql, dims, preferred_element_type=F32)
        + lax.dot_general(kl, qh, dims, preferred_element_type=F32)
    )
    c = c_ref[:, pl.ds(t * t_rows, t_rows)]
    smax = jnp.max(jnp.where(c > 0.0, st, -1e30), axis=0, keepdims=True)
    ssum = jnp.sum(st * c, axis=0, keepdims=True)
    m_ref[0, :, pl.ds(t * t_rows, t_rows)] = smax - ssum / l_k


def _topk_body(m_ref, oh_ref, idx_ref, *, u, l_q):
    m = m_ref[0]
    iota_r = lax.broadcasted_iota(jnp.int32, (1, l_q), 1)

    def body(j, mcur):
        mx = jnp.max(mcur)
        amax = jnp.min(jnp.where(mcur == mx, iota_r, l_q))
        oh_ref[0, pl.ds(j, 1), :] = (iota_r == amax).astype(F32)
        idx_ref[0, pl.ds(j, 1), :] = amax.astype(jnp.int32).reshape(1, 1)
        return jnp.where(iota_r == amax, -1e30, mcur)

    lax.fori_loop(0, u, body, m)


def _softmax_rows_body(oh_ref, qh_ref, ql_ref, kh_ref, kl_ref, p_ref, *, scale):
    sel = (((1,), (0,)), ((), ()))
    dims = (((1,), (1,)), ((), ()))
    oh16 = oh_ref[0].astype(BF16)
    qred = (
        lax.dot_general(oh16, qh_ref[0, 0], sel, preferred_element_type=F32)
        + lax.dot_general(oh16, ql_ref[0, 0], sel, preferred_element_type=F32)
    )
    qrh = qred.astype(BF16)
    qrl = (qred - qrh.astype(F32)).astype(BF16)
    kh = kh_ref[0, 0]
    kl = kl_ref[0, 0]
    s = (
        lax.dot_general(qrh, kh, dims, preferred_element_type=F32)
        + lax.dot_general(qrh, kl, dims, preferred_element_type=F32)
        + lax.dot_general(qrl, kh, dims, preferred_element_type=F32)
    ) * scale
    p = jnp.exp(s - jnp.max(s, axis=1, keepdims=True))
    p_ref[0] = p / jnp.sum(p, axis=1, keepdims=True)


def _merge_body(oh_ref, p_ref, v_ref, pcomb_ref, canon_ref, orows_ref,
                *, n_head, u, l_k):
    hu = n_head * u
    pall = p_ref[0]
    ohb = oh_ref[0]
    eq = lax.dot_general(ohb, ohb, (((1,), (1,)), ((), ())),
                         precision=_PREC, preferred_element_type=F32)
    cnt = jnp.sum(eq, axis=1, keepdims=True)
    ii = lax.broadcasted_iota(jnp.int32, (hu, hu), 0)
    jj = lax.broadcasted_iota(jnp.int32, (hu, hu), 1)
    prior = jnp.sum(eq * (jj < ii).astype(F32), axis=1, keepdims=True)
    canon = (prior == 0.0).astype(F32)
    base = (n_head - cnt) / (n_head * l_k)
    pc = base + lax.dot_general(eq, pall, (((1,), (0,)), ((), ())),
                                precision=_PREC, preferred_element_type=F32) / n_head
    pcomb_ref[0] = pc
    canon_ref[0] = canon
    orows_ref[0] = lax.dot_general(pc * canon, v_ref[0], (((1,), (0,)), ((), ())),
                                   precision=_PREC, preferred_element_type=F32)


def _attn_body(oh_ref, pc_ref, canon_ref, attn_ref, *, n_head, t_rows, l_k):
    t = pl.program_id(1)
    ohd = oh_ref[0, :, pl.ds(t * t_rows, t_rows)] * canon_ref[0]
    pd = pc_ref[0] * canon_ref[0]
    content = lax.dot_general(ohd, pd, (((0,), (0,)), ((), ())),
                              precision=_PREC, preferred_element_type=F32)
    selrow = lax.dot_general(ohd, jnp.ones((ohd.shape[0], 1), F32),
                             (((0,), (0,)), ((), ())),
                             precision=_PREC, preferred_element_type=F32)
    attn_ref[0] = content + (1.0 - selrow) * (1.0 / l_k)


def _out_body(oh_ref, canon_ref, orows_ref, v_ref, wo_ref, bo_ref, out_ref,
              *, n_head, t_rows, l_k):
    t = pl.program_id(1)
    ohd = oh_ref[0, :, pl.ds(t * t_rows, t_rows)] * canon_ref[0]
    meanv = jnp.sum(v_ref[0], axis=0, keepdims=True) / l_k
    selrow = lax.dot_general(ohd, jnp.ones((ohd.shape[0], 1), F32),
                             (((0,), (0,)), ((), ())),
                             precision=_PREC, preferred_element_type=F32)
    opre = lax.dot_general(ohd, orows_ref[0], (((0,), (0,)), ((), ())),
                           precision=_PREC, preferred_element_type=F32)
    opre = opre + (1.0 - selrow) * meanv
    y = lax.dot_general(opre, wo_ref[...], (((1,), (1,)), ((), ())),
                        precision=_PREC, preferred_element_type=F32) + bo_ref[...]
    out_ref[0] = y


def kernel(q, k, v, mask, Wq, bq, Wk, bk, Wv, bv, Wo, bo):
    del mask
    b_sz, l_q, d_model = q.shape
    l_k = k.shape[1]
    n_head = 12
    dk = d_model // n_head
    dv = Wv.shape[0]
    u = min(int(3 * np.ceil(np.log(l_q))), l_q)
    hu = n_head * u
    scale = 1.0 / sqrt(dk)
    t_rows = 256

    cnts_t, _ = _sample_counts_t(l_q, l_k)

    qp_hi, qp_lo = _project(q.reshape(b_sz * l_q, d_model), Wq, bq, 512,
                            split=True)
    kp_hi, kp_lo = _project(k.reshape(b_sz * l_k, d_model), Wk, bk, 512,
                            split=True)
    vp = _project(v.reshape(b_sz * l_k, d_model), Wv, bv, 512)
    tr = lambda a, l: a.reshape(b_sz, l, n_head, dk).transpose(0, 2, 1, 3)
    q4h = tr(qp_hi, l_q)
    q4l = tr(qp_lo, l_q)
    k4h = tr(kp_hi, l_k)
    k4l = tr(kp_lo, l_k)
    v3 = vp.reshape(b_sz, l_k, dv)

    m = pl.pallas_call(
        functools.partial(_m_body, t_rows=t_rows, l_k=l_k),
        grid=(b_sz * n_head, l_q // t_rows),
        in_specs=[
            pl.BlockSpec((1, 1, t_rows, dk),
                         lambda bh, t, H=n_head: (bh // H, bh % H, t, 0)),
            pl.BlockSpec((1, 1, t_rows, dk),
                         lambda bh, t, H=n_head: (bh // H, bh % H, t, 0)),
            pl.BlockSpec((1, 1, l_k, dk),
                         lambda bh, t, H=n_head: (bh // H, bh % H, 0, 0)),
            pl.BlockSpec((1, 1, l_k, dk),
                         lambda bh, t, H=n_head: (bh // H, bh % H, 0, 0)),
            pl.BlockSpec((l_k, l_q), lambda bh, t: (0, 0)),
        ],
        out_specs=pl.BlockSpec((1, 1, l_q), lambda bh, t: (bh, 0, 0)),
        out_shape=jax.ShapeDtypeStruct((b_sz * n_head, 1, l_q), F32),
    )(q4h, q4l, k4h, k4l, cnts_t)

    oh, top_idx = pl.pallas_call(
        functools.partial(_topk_body, u=u, l_q=l_q),
        grid=(b_sz * n_head,),
        in_specs=[pl.BlockSpec((1, 1, l_q), lambda bh: (bh, 0, 0))],
        out_specs=[
            pl.BlockSpec((1, u, l_q), lambda bh: (bh, 0, 0)),
            pl.BlockSpec((1, u, 1), lambda bh: (bh, 0, 0)),
        ],
        out_shape=[
            jax.ShapeDtypeStruct((b_sz * n_head, u, l_q), F32),
            jax.ShapeDtypeStruct((b_sz * n_head, u, 1), jnp.int32),
        ],
    )(m)

    pall = pl.pallas_call(
        functools.partial(_softmax_rows_body, scale=scale),
        grid=(b_sz * n_head,),
        in_specs=[
            pl.BlockSpec((1, u, l_q), lambda bh: (bh, 0, 0)),
            pl.BlockSpec((1, 1, l_q, dk),
                         lambda bh, H=n_head: (bh // H, bh % H, 0, 0)),
            pl.BlockSpec((1, 1, l_q, dk),
                         lambda bh, H=n_head: (bh // H, bh % H, 0, 0)),
            pl.BlockSpec((1, 1, l_k, dk),
                         lambda bh, H=n_head: (bh // H, bh % H, 0, 0)),
            pl.BlockSpec((1, 1, l_k, dk),
                         lambda bh, H=n_head: (bh // H, bh % H, 0, 0)),
        ],
        out_specs=pl.BlockSpec((1, u, l_k), lambda bh: (bh, 0, 0)),
        out_shape=jax.ShapeDtypeStruct((b_sz * n_head, u, l_k), F32),
    )(oh, q4h, q4l, k4h, k4l)

    ohb3 = oh.reshape(b_sz, hu, l_q)
    pall3 = pall.reshape(b_sz, hu, l_k)

    pcomb, canon, orows = pl.pallas_call(
        functools.partial(_merge_body, n_head=n_head, u=u, l_k=l_k),
        grid=(b_sz,),
        in_specs=[
            pl.BlockSpec((1, hu, l_q), lambda b: (b, 0, 0)),
            pl.BlockSpec((1, hu, l_k), lambda b: (b, 0, 0)),
            pl.BlockSpec((1, l_k, dv), lambda b: (b, 0, 0)),
        ],
        out_specs=[
            pl.BlockSpec((1, hu, l_k), lambda b: (b, 0, 0)),
            pl.BlockSpec((1, hu, 1), lambda b: (b, 0, 0)),
            pl.BlockSpec((1, hu, dv), lambda b: (b, 0, 0)),
        ],
        out_shape=[
            jax.ShapeDtypeStruct((b_sz, hu, l_k), F32),
            jax.ShapeDtypeStruct((b_sz, hu, 1), F32),
            jax.ShapeDtypeStruct((b_sz, hu, dv), F32),
        ],
    )(ohb3, pall3, v3)

    attn = pl.pallas_call(
        functools.partial(_attn_body, n_head=n_head, t_rows=t_rows, l_k=l_k),
        grid=(b_sz, l_q // t_rows),
        in_specs=[
            pl.BlockSpec((1, hu, l_q), lambda b, t: (b, 0, 0)),
            pl.BlockSpec((1, hu, l_k), lambda b, t: (b, 0, 0)),
            pl.BlockSpec((1, hu, 1), lambda b, t: (b, 0, 0)),
        ],
        out_specs=pl.BlockSpec((1, t_rows, l_k), lambda b, t: (b, t, 0)),
        out_shape=jax.ShapeDtypeStruct((b_sz, l_q, l_k), F32),
    )(ohb3, pcomb, canon)

    out = pl.pallas_call(
        functools.partial(_out_body, n_head=n_head, t_rows=t_rows, l_k=l_k),
        grid=(b_sz, l_q // t_rows),
        in_specs=[
            pl.BlockSpec((1, hu, l_q), lambda b, t: (b, 0, 0)),
            pl.BlockSpec((1, hu, 1), lambda b, t: (b, 0, 0)),
            pl.BlockSpec((1, hu, dv), lambda b, t: (b, 0, 0)),
            pl.BlockSpec((1, l_k, dv), lambda b, t: (b, 0, 0)),
            pl.BlockSpec((d_model, dv), lambda b, t: (0, 0)),
            pl.BlockSpec((1, d_model), lambda b, t: (0, 0)),
        ],
        out_specs=pl.BlockSpec((1, t_rows, d_model), lambda b, t: (b, t, 0)),
        out_shape=jax.ShapeDtypeStruct((b_sz, l_q, d_model), F32),
    )(ohb3, canon, orows, v3, Wo, bo.reshape(1, d_model))

    return (out, attn)


# speedup vs baseline: 1.3257x; 1.3045x over previous
import functools
from math import sqrt

import numpy as np
import jax
import jax.numpy as jnp
from jax import lax
from jax.experimental import pallas as pl
from jax.experimental.pallas import tpu as pltpu

F32 = jnp.float32
_PREC = lax.Precision.HIGHEST
BF16 = jnp.bfloat16

_CONSTS = {}


def _threefry2x32(k0, k1, c0, c1):
    k0, k1 = np.uint32(k0), np.uint32(k1)
    x0 = (c0 + k0).astype(np.uint32)
    x1 = (c1 + k1).astype(np.uint32)
    ks = [k0, k1, np.uint32(np.uint32(k0) ^ np.uint32(k1) ^ np.uint32(0x1BD11BDA))]
    rots = [[13, 15, 26, 6], [17, 29, 16, 24]]
    for g in range(5):
        for r in rots[g % 2]:
            x0 = (x0 + x1).astype(np.uint32)
            x1 = ((x1 << np.uint32(r)) | (x1 >> np.uint32(32 - r))).astype(np.uint32) ^ x0
        x0 = (x0 + ks[(g + 1) % 3]).astype(np.uint32)
        x1 = (x1 + ks[(g + 2) % 3] + np.uint32(g + 1)).astype(np.uint32)
    return x0, x1


def _np_randint(shape, span):
    o0, o1 = _threefry2x32(np.uint32(0), np.uint32(42),
                           np.zeros(2, np.uint32), np.arange(2, dtype=np.uint32))
    n = int(np.prod(shape))

    def bits(sk0, sk1):
        c = np.arange(n, dtype=np.uint64)
        hi = (c >> np.uint64(32)).astype(np.uint32)
        lo = (c & np.uint64(0xFFFFFFFF)).astype(np.uint32)
        x0, x1 = _threefry2x32(sk0, sk1, hi, lo)
        return (x0 ^ x1).astype(np.uint32)

    u = bits(o0[0], o1[0])
    v = bits(o0[1], o1[1])
    be = np.uint32(span)
    bh = np.uint32((np.uint64(65536 % span) ** 2) % np.uint64(span))
    out = ((u % be) * bh + (v % be)) % be
    return out.astype(np.int32).reshape(shape)


def _sample_counts_t(l_q: int, l_k: int):
    ck = (l_q, l_k)
    if ck not in _CONSTS:
        u_part = min(int(3 * np.ceil(np.log(l_k))), l_k)
        idx_np = _np_randint((l_q, u_part), l_k)
        cnt = np.zeros((l_k, l_q), dtype=np.float32)
        rows = np.broadcast_to(np.arange(l_q)[:, None], idx_np.shape)
        np.add.at(cnt, (idx_np, rows), 1.0)
        neg = np.where(cnt > 0.0, 0.0, -1e30).astype(np.float32)
        _CONSTS[ck] = (jnp.asarray(cnt), jnp.asarray(neg))
    return _CONSTS[ck]


def _proj_body(x_ref, w_ref, b_ref, o_ref):
    o_ref[...] = (
        lax.dot_general(
            x_ref[...], w_ref[...], (((1,), (1,)), ((), ())),
            precision=_PREC, preferred_element_type=F32,
        )
        + b_ref[...]
    )


def _proj_split_body(x_ref, w_ref, b_ref, hi_ref, lo_ref):
    y = (
        lax.dot_general(
            x_ref[...], w_ref[...], (((1,), (1,)), ((), ())),
            precision=_PREC, preferred_element_type=F32,
        )
        + b_ref[...]
    )
    hi = y.astype(BF16)
    hi_ref[...] = hi
    lo_ref[...] = (y - hi.astype(F32)).astype(BF16)


def _project(x2d, w, b, tile, split=False):
    n, d_in = x2d.shape
    d_out = w.shape[0]
    in_specs = [
        pl.BlockSpec((tile, d_in), lambda i: (i, 0)),
        pl.BlockSpec((d_out, d_in), lambda i: (0, 0)),
        pl.BlockSpec((1, d_out), lambda i: (0, 0)),
    ]
    if not split:
        return pl.pallas_call(
            _proj_body,
            grid=(n // tile,),
            in_specs=in_specs,
            out_specs=pl.BlockSpec((tile, d_out), lambda i: (i, 0)),
            out_shape=jax.ShapeDtypeStruct((n, d_out), F32),
        )(x2d, w, b.reshape(1, d_out))
    return pl.pallas_call(
        _proj_split_body,
        grid=(n // tile,),
        in_specs=in_specs,
        out_specs=[
            pl.BlockSpec((tile, d_out), lambda i: (i, 0)),
            pl.BlockSpec((tile, d_out), lambda i: (i, 0)),
        ],
        out_shape=[
            jax.ShapeDtypeStruct((n, d_out), BF16),
            jax.ShapeDtypeStruct((n, d_out), BF16),
        ],
    )(x2d, w, b.reshape(1, d_out))


def _m_body(qh_ref, ql_ref, kh_ref, kl_ref, c_ref, m_ref, *, t_rows, l_k):
    t = pl.program_id(1)
    dims = (((1,), (1,)), ((), ()))
    qh = qh_ref[0, 0]
    ql = ql_ref[0, 0]
    kh = kh_ref[0, 0]
    kl = kl_ref[0, 0]
    st = (
        lax.dot_general(kh, qh, dims, preferred_element_type=F32)
        + lax.dot_general(kh, ql, dims, preferred_element_type=F32)
        + lax.dot_general(kl, qh, dims, preferred_element_type=F32)
    )
    c = c_ref[:, pl.ds(t * t_rows, t_rows)]
    smax = jnp.max(jnp.where(c > 0.0, st, -1e30), axis=0, keepdims=True)
    ssum = jnp.sum(st * c, axis=0, keepdims=True)
    m_ref[0, :, pl.ds(t * t_rows, t_rows)] = smax - ssum / l_k


def _topk_body(m_ref, oh_ref, idx_ref, *, u, l_q, rows):
    m = m_ref[:, 0, :]
    iota_q = lax.broadcasted_iota(jnp.int32, (rows, l_q), 1)
    iota_u = lax.broadcasted_iota(jnp.int32, (rows, u), 1)

    def body(j, carry):
        mcur, idx = carry
        mx = jnp.max(mcur, axis=1, keepdims=True)
        amax = jnp.min(jnp.where(mcur == mx, iota_q, l_q),
                       axis=1, keepdims=True)
        idx = jnp.where(iota_u == j, amax, idx)
        mcur = jnp.where(iota_q == amax, -1e30, mcur)
        return mcur, idx

    _, idx = lax.fori_loop(0, u, body,
                           (m, jnp.zeros((rows, u), jnp.int32)))
    iota3 = lax.broadcasted_iota(jnp.int32, (rows, u, l_q), 2)
    oh_ref[...] = (idx[:, :, None] == iota3).astype(F32)
    idx_ref[...] = idx[:, :, None]


def _softmax_rows_body(oh_ref, qh_ref, ql_ref, kh_ref, kl_ref, p_ref, *, scale):
    sel = (((1,), (0,)), ((), ()))
    dims = (((1,), (1,)), ((), ()))
    oh16 = oh_ref[0].astype(BF16)
    qred = (
        lax.dot_general(oh16, qh_ref[0, 0], sel, preferred_element_type=F32)
        + lax.dot_general(oh16, ql_ref[0, 0], sel, preferred_element_type=F32)
    )
    qrh = qred.astype(BF16)
    qrl = (qred - qrh.astype(F32)).astype(BF16)
    kh = kh_ref[0, 0]
    kl = kl_ref[0, 0]
    s = (
        lax.dot_general(qrh, kh, dims, preferred_element_type=F32)
        + lax.dot_general(qrh, kl, dims, preferred_element_type=F32)
        + lax.dot_general(qrl, kh, dims, preferred_element_type=F32)
    ) * scale
    p = jnp.exp(s - jnp.max(s, axis=1, keepdims=True))
    p_ref[0] = p / jnp.sum(p, axis=1, keepdims=True)


def _merge_body(oh_ref, p_ref, v_ref, pcomb_ref, canon_ref, orows_ref,
                *, n_head, u, l_k):
    hu = n_head * u
    pall = p_ref[0]
    ohb = oh_ref[0]
    eq = lax.dot_general(ohb, ohb, (((1,), (1,)), ((), ())),
                         precision=_PREC, preferred_element_type=F32)
    cnt = jnp.sum(eq, axis=1, keepdims=True)
    ii = lax.broadcasted_iota(jnp.int32, (hu, hu), 0)
    jj = lax.broadcasted_iota(jnp.int32, (hu, hu), 1)
    prior = jnp.sum(eq * (jj < ii).astype(F32), axis=1, keepdims=True)
    canon = (prior == 0.0).astype(F32)
    base = (n_head - cnt) / (n_head * l_k)
    pc = base + lax.dot_general(eq, pall, (((1,), (0,)), ((), ())),
                                precision=_PREC, preferred_element_type=F32) / n_head
    pcomb_ref[0] = pc
    canon_ref[0] = canon
    orows_ref[0] = lax.dot_general(pc * canon, v_ref[0], (((1,), (0,)), ((), ())),
                                   precision=_PREC, preferred_element_type=F32)


def _attn_body(oh_ref, pc_ref, canon_ref, attn_ref, *, n_head, t_rows, l_k):
    t = pl.program_id(1)
    ohd = oh_ref[0, :, pl.ds(t * t_rows, t_rows)] * canon_ref[0]
    pd = pc_ref[0] * canon_ref[0]
    content = lax.dot_general(ohd, pd, (((0,), (0,)), ((), ())),
                              precision=_PREC, preferred_element_type=F32)
    selrow = lax.dot_general(ohd, jnp.ones((ohd.shape[0], 1), F32),
                             (((0,), (0,)), ((), ())),
                             precision=_PREC, preferred_element_type=F32)
    attn_ref[0] = content + (1.0 - selrow) * (1.0 / l_k)


def _out_body(oh_ref, canon_ref, orows_ref, v_ref, wo_ref, bo_ref, out_ref,
              *, n_head, t_rows, l_k):
    t = pl.program_id(1)
    ohd = oh_ref[0, :, pl.ds(t * t_rows, t_rows)] * canon_ref[0]
    meanv = jnp.sum(v_ref[0], axis=0, keepdims=True) / l_k
    selrow = lax.dot_general(ohd, jnp.ones((ohd.shape[0], 1), F32),
                             (((0,), (0,)), ((), ())),
                             precision=_PREC, preferred_element_type=F32)
    opre = lax.dot_general(ohd, orows_ref[0], (((0,), (0,)), ((), ())),
                           precision=_PREC, preferred_element_type=F32)
    opre = opre + (1.0 - selrow) * meanv
    y = lax.dot_general(opre, wo_ref[...], (((1,), (1,)), ((), ())),
                        precision=_PREC, preferred_element_type=F32) + bo_ref[...]
    out_ref[0] = y


def kernel(q, k, v, mask, Wq, bq, Wk, bk, Wv, bv, Wo, bo):
    del mask
    b_sz, l_q, d_model = q.shape
    l_k = k.shape[1]
    n_head = 12
    dk = d_model // n_head
    dv = Wv.shape[0]
    u = min(int(3 * np.ceil(np.log(l_q))), l_q)
    hu = n_head * u
    scale = 1.0 / sqrt(dk)
    t_rows = 256

    cnts_t, _ = _sample_counts_t(l_q, l_k)

    qp_hi, qp_lo = _project(q.reshape(b_sz * l_q, d_model), Wq, bq, 512,
                            split=True)
    kp_hi, kp_lo = _project(k.reshape(b_sz * l_k, d_model), Wk, bk, 512,
                            split=True)
    vp = _project(v.reshape(b_sz * l_k, d_model), Wv, bv, 512)
    tr = lambda a, l: a.reshape(b_sz, l, n_head, dk).transpose(0, 2, 1, 3)
    q4h = tr(qp_hi, l_q)
    q4l = tr(qp_lo, l_q)
    k4h = tr(kp_hi, l_k)
    k4l = tr(kp_lo, l_k)
    v3 = vp.reshape(b_sz, l_k, dv)

    m = pl.pallas_call(
        functools.partial(_m_body, t_rows=t_rows, l_k=l_k),
        grid=(b_sz * n_head, l_q // t_rows),
        in_specs=[
            pl.BlockSpec((1, 1, t_rows, dk),
                         lambda bh, t, H=n_head: (bh // H, bh % H, t, 0)),
            pl.BlockSpec((1, 1, t_rows, dk),
                         lambda bh, t, H=n_head: (bh // H, bh % H, t, 0)),
            pl.BlockSpec((1, 1, l_k, dk),
                         lambda bh, t, H=n_head: (bh // H, bh % H, 0, 0)),
            pl.BlockSpec((1, 1, l_k, dk),
                         lambda bh, t, H=n_head: (bh // H, bh % H, 0, 0)),
            pl.BlockSpec((l_k, l_q), lambda bh, t: (0, 0)),
        ],
        out_specs=pl.BlockSpec((1, 1, l_q), lambda bh, t: (bh, 0, 0)),
        out_shape=jax.ShapeDtypeStruct((b_sz * n_head, 1, l_q), F32),
    )(q4h, q4l, k4h, k4l, cnts_t)

    bh_rows = b_sz * n_head
    oh, top_idx = pl.pallas_call(
        functools.partial(_topk_body, u=u, l_q=l_q, rows=bh_rows),
        grid=(1,),
        in_specs=[pl.BlockSpec((bh_rows, 1, l_q), lambda i: (0, 0, 0))],
        out_specs=[
            pl.BlockSpec((bh_rows, u, l_q), lambda i: (0, 0, 0)),
            pl.BlockSpec((bh_rows, u, 1), lambda i: (0, 0, 0)),
        ],
        out_shape=[
            jax.ShapeDtypeStruct((bh_rows, u, l_q), F32),
            jax.ShapeDtypeStruct((bh_rows, u, 1), jnp.int32),
        ],
    )(m)

    pall = pl.pallas_call(
        functools.partial(_softmax_rows_body, scale=scale),
        grid=(b_sz * n_head,),
        in_specs=[
            pl.BlockSpec((1, u, l_q), lambda bh: (bh, 0, 0)),
            pl.BlockSpec((1, 1, l_q, dk),
                         lambda bh, H=n_head: (bh // H, bh % H, 0, 0)),
            pl.BlockSpec((1, 1, l_q, dk),
                         lambda bh, H=n_head: (bh // H, bh % H, 0, 0)),
            pl.BlockSpec((1, 1, l_k, dk),
                         lambda bh, H=n_head: (bh // H, bh % H, 0, 0)),
            pl.BlockSpec((1, 1, l_k, dk),
                         lambda bh, H=n_head: (bh // H, bh % H, 0, 0)),
        ],
        out_specs=pl.BlockSpec((1, u, l_k), lambda bh: (bh, 0, 0)),
        out_shape=jax.ShapeDtypeStruct((b_sz * n_head, u, l_k), F32),
    )(oh, q4h, q4l, k4h, k4l)

    ohb3 = oh.reshape(b_sz, hu, l_q)
    pall3 = pall.reshape(b_sz, hu, l_k)

    pcomb, canon, orows = pl.pallas_call(
        functools.partial(_merge_body, n_head=n_head, u=u, l_k=l_k),
        grid=(b_sz,),
        in_specs=[
            pl.BlockSpec((1, hu, l_q), lambda b: (b, 0, 0)),
            pl.BlockSpec((1, hu, l_k), lambda b: (b, 0, 0)),
            pl.BlockSpec((1, l_k, dv), lambda b: (b, 0, 0)),
        ],
        out_specs=[
            pl.BlockSpec((1, hu, l_k), lambda b: (b, 0, 0)),
            pl.BlockSpec((1, hu, 1), lambda b: (b, 0, 0)),
            pl.BlockSpec((1, hu, dv), lambda b: (b, 0, 0)),
        ],
        out_shape=[
            jax.ShapeDtypeStruct((b_sz, hu, l_k), F32),
            jax.ShapeDtypeStruct((b_sz, hu, 1), F32),
            jax.ShapeDtypeStruct((b_sz, hu, dv), F32),
        ],
    )(ohb3, pall3, v3)

    attn = pl.pallas_call(
        functools.partial(_attn_body, n_head=n_head, t_rows=t_rows, l_k=l_k),
        grid=(b_sz, l_q // t_rows),
        in_specs=[
            pl.BlockSpec((1, hu, l_q), lambda b, t: (b, 0, 0)),
            pl.BlockSpec((1, hu, l_k), lambda b, t: (b, 0, 0)),
            pl.BlockSpec((1, hu, 1), lambda b, t: (b, 0, 0)),
        ],
        out_specs=pl.BlockSpec((1, t_rows, l_k), lambda b, t: (b, t, 0)),
        out_shape=jax.ShapeDtypeStruct((b_sz, l_q, l_k), F32),
    )(ohb3, pcomb, canon)

    out = pl.pallas_call(
        functools.partial(_out_body, n_head=n_head, t_rows=t_rows, l_k=l_k),
        grid=(b_sz, l_q // t_rows),
        in_specs=[
            pl.BlockSpec((1, hu, l_q), lambda b, t: (b, 0, 0)),
            pl.BlockSpec((1, hu, 1), lambda b, t: (b, 0, 0)),
            pl.BlockSpec((1, hu, dv), lambda b, t: (b, 0, 0)),
            pl.BlockSpec((1, l_k, dv), lambda b, t: (b, 0, 0)),
            pl.BlockSpec((d_model, dv), lambda b, t: (0, 0)),
            pl.BlockSpec((1, d_model), lambda b, t: (0, 0)),
        ],
        out_specs=pl.BlockSpec((1, t_rows, d_model), lambda b, t: (b, t, 0)),
        out_shape=jax.ShapeDtypeStruct((b_sz, l_q, d_model), F32),
    )(ohb3, canon, orows, v3, Wo, bo.reshape(1, d_model))

    return (out, attn)


# speedup vs baseline: 1.3458x; 1.0152x over previous
import functools
from math import sqrt

import numpy as np
import jax
import jax.numpy as jnp
from jax import lax
from jax.experimental import pallas as pl
from jax.experimental.pallas import tpu as pltpu

F32 = jnp.float32
_PREC = lax.Precision.HIGHEST
BF16 = jnp.bfloat16

_CONSTS = {}


def _threefry2x32(k0, k1, c0, c1):
    k0, k1 = np.uint32(k0), np.uint32(k1)
    x0 = (c0 + k0).astype(np.uint32)
    x1 = (c1 + k1).astype(np.uint32)
    ks = [k0, k1, np.uint32(np.uint32(k0) ^ np.uint32(k1) ^ np.uint32(0x1BD11BDA))]
    rots = [[13, 15, 26, 6], [17, 29, 16, 24]]
    for g in range(5):
        for r in rots[g % 2]:
            x0 = (x0 + x1).astype(np.uint32)
            x1 = ((x1 << np.uint32(r)) | (x1 >> np.uint32(32 - r))).astype(np.uint32) ^ x0
        x0 = (x0 + ks[(g + 1) % 3]).astype(np.uint32)
        x1 = (x1 + ks[(g + 2) % 3] + np.uint32(g + 1)).astype(np.uint32)
    return x0, x1


def _np_randint(shape, span):
    o0, o1 = _threefry2x32(np.uint32(0), np.uint32(42),
                           np.zeros(2, np.uint32), np.arange(2, dtype=np.uint32))
    n = int(np.prod(shape))

    def bits(sk0, sk1):
        c = np.arange(n, dtype=np.uint64)
        hi = (c >> np.uint64(32)).astype(np.uint32)
        lo = (c & np.uint64(0xFFFFFFFF)).astype(np.uint32)
        x0, x1 = _threefry2x32(sk0, sk1, hi, lo)
        return (x0 ^ x1).astype(np.uint32)

    u = bits(o0[0], o1[0])
    v = bits(o0[1], o1[1])
    be = np.uint32(span)
    bh = np.uint32((np.uint64(65536 % span) ** 2) % np.uint64(span))
    out = ((u % be) * bh + (v % be)) % be
    return out.astype(np.int32).reshape(shape)


def _sample_counts_t(l_q: int, l_k: int):
    ck = (l_q, l_k)
    if ck not in _CONSTS:
        u_part = min(int(3 * np.ceil(np.log(l_k))), l_k)
        idx_np = _np_randint((l_q, u_part), l_k)
        cnt = np.zeros((l_k, l_q), dtype=np.float32)
        rows = np.broadcast_to(np.arange(l_q)[:, None], idx_np.shape)
        np.add.at(cnt, (idx_np, rows), 1.0)
        neg = np.where(cnt > 0.0, 0.0, -1e30).astype(np.float32)
        _CONSTS[ck] = (jnp.asarray(cnt), jnp.asarray(neg))
    return _CONSTS[ck]


def _proj_body(x_ref, w_ref, b_ref, o_ref):
    o_ref[...] = (
        lax.dot_general(
            x_ref[...], w_ref[...], (((1,), (1,)), ((), ())),
            precision=_PREC, preferred_element_type=F32,
        )
        + b_ref[...]
    )


def _proj_split_body(x_ref, w_ref, b_ref, hi_ref, lo_ref):
    y = (
        lax.dot_general(
            x_ref[...], w_ref[...], (((1,), (1,)), ((), ())),
            precision=_PREC, preferred_element_type=F32,
        )
        + b_ref[...]
    )
    hi = y.astype(BF16)
    hi_ref[...] = hi
    lo_ref[...] = (y - hi.astype(F32)).astype(BF16)


def _project(x2d, w, b, tile, split=False):
    n, d_in = x2d.shape
    d_out = w.shape[0]
    in_specs = [
        pl.BlockSpec((tile, d_in), lambda i: (i, 0)),
        pl.BlockSpec((d_out, d_in), lambda i: (0, 0)),
        pl.BlockSpec((1, d_out), lambda i: (0, 0)),
    ]
    if not split:
        return pl.pallas_call(
            _proj_body,
            grid=(n // tile,),
            in_specs=in_specs,
            out_specs=pl.BlockSpec((tile, d_out), lambda i: (i, 0)),
            out_shape=jax.ShapeDtypeStruct((n, d_out), F32),
        )(x2d, w, b.reshape(1, d_out))
    return pl.pallas_call(
        _proj_split_body,
        grid=(n // tile,),
        in_specs=in_specs,
        out_specs=[
            pl.BlockSpec((tile, d_out), lambda i: (i, 0)),
            pl.BlockSpec((tile, d_out), lambda i: (i, 0)),
        ],
        out_shape=[
            jax.ShapeDtypeStruct((n, d_out), BF16),
            jax.ShapeDtypeStruct((n, d_out), BF16),
        ],
    )(x2d, w, b.reshape(1, d_out))


def _m_body(qh_ref, ql_ref, kh_ref, kl_ref, c_ref, m_ref, *, t_rows, l_k):
    t = pl.program_id(1)
    dims = (((1,), (1,)), ((), ()))
    qh = qh_ref[0, 0]
    ql = ql_ref[0, 0]
    kh = kh_ref[0, 0]
    kl = kl_ref[0, 0]
    st = (
        lax.dot_general(kh, qh, dims, preferred_element_type=F32)
        + lax.dot_general(kh, ql, dims, preferred_element_type=F32)
        + lax.dot_general(kl, qh, dims, preferred_element_type=F32)
    )
    c = c_ref[:, pl.ds(t * t_rows, t_rows)]
    smax = jnp.max(jnp.where(c > 0.0, st, -1e30), axis=0, keepdims=True)
    ssum = jnp.sum(st * c, axis=0, keepdims=True)
    m_ref[0, :, pl.ds(t * t_rows, t_rows)] = smax - ssum / l_k


def _topk_body(m_ref, oh_ref, idx_ref, *, u, l_q, rows):
    m = m_ref[:, 0, :]
    iota_q = lax.broadcasted_iota(jnp.int32, (rows, l_q), 1)
    iota_u = lax.broadcasted_iota(jnp.int32, (rows, u), 1)

    def body(j, carry):
        mcur, idx = carry
        mx = jnp.max(mcur, axis=1, keepdims=True)
        amax = jnp.min(jnp.where(mcur == mx, iota_q, l_q),
                       axis=1, keepdims=True)
        idx = jnp.where(iota_u == j, amax, idx)
        mcur = jnp.where(iota_q == amax, -1e30, mcur)
        return mcur, idx

    _, idx = lax.fori_loop(0, u, body,
                           (m, jnp.zeros((rows, u), jnp.int32)))
    iota3 = lax.broadcasted_iota(jnp.int32, (rows, u, l_q), 2)
    oh_ref[...] = (idx[:, :, None] == iota3).astype(F32)
    idx_ref[...] = idx[:, :, None]


def _softmax_rows_body(oh_ref, qh_ref, ql_ref, kh_ref, kl_ref, p_ref, *, scale):
    sel = (((1,), (0,)), ((), ()))
    dims = (((1,), (1,)), ((), ()))
    oh16 = oh_ref[0].astype(BF16)
    qred = (
        lax.dot_general(oh16, qh_ref[0, 0], sel, preferred_element_type=F32)
        + lax.dot_general(oh16, ql_ref[0, 0], sel, preferred_element_type=F32)
    )
    qrh = qred.astype(BF16)
    qrl = (qred - qrh.astype(F32)).astype(BF16)
    kh = kh_ref[0, 0]
    kl = kl_ref[0, 0]
    s = (
        lax.dot_general(qrh, kh, dims, preferred_element_type=F32)
        + lax.dot_general(qrh, kl, dims, preferred_element_type=F32)
        + lax.dot_general(qrl, kh, dims, preferred_element_type=F32)
    ) * scale
    p = jnp.exp(s - jnp.max(s, axis=1, keepdims=True))
    p_ref[0] = p / jnp.sum(p, axis=1, keepdims=True)


def _merge_body(oh_ref, p_ref, v_ref, pcomb_ref, canon_ref, orows_ref,
                *, n_head, u, l_k):
    hu = n_head * u
    pall = p_ref[0]
    ohb = oh_ref[0]
    eq = lax.dot_general(ohb, ohb, (((1,), (1,)), ((), ())),
                         precision=_PREC, preferred_element_type=F32)
    cnt = jnp.sum(eq, axis=1, keepdims=True)
    ii = lax.broadcasted_iota(jnp.int32, (hu, hu), 0)
    jj = lax.broadcasted_iota(jnp.int32, (hu, hu), 1)
    prior = jnp.sum(eq * (jj < ii).astype(F32), axis=1, keepdims=True)
    canon = (prior == 0.0).astype(F32)
    base = (n_head - cnt) / (n_head * l_k)
    pc = base + lax.dot_general(eq, pall, (((1,), (0,)), ((), ())),
                                precision=_PREC, preferred_element_type=F32) / n_head
    pcomb_ref[0] = pc
    canon_ref[0] = canon
    orows_ref[0] = lax.dot_general(pc * canon, v_ref[0], (((1,), (0,)), ((), ())),
                                   precision=_PREC, preferred_element_type=F32)


def _attn_body(oh_ref, pc_ref, canon_ref, attn_ref, *, n_head, t_rows, l_k):
    t = pl.program_id(1)
    ohd = oh_ref[0, :, pl.ds(t * t_rows, t_rows)] * canon_ref[0]
    pd = pc_ref[0] * canon_ref[0]
    content = lax.dot_general(ohd, pd, (((0,), (0,)), ((), ())),
                              precision=_PREC, preferred_element_type=F32)
    selrow = lax.dot_general(ohd, jnp.ones((ohd.shape[0], 1), F32),
                             (((0,), (0,)), ((), ())),
                             precision=_PREC, preferred_element_type=F32)
    attn_ref[0] = content + (1.0 - selrow) * (1.0 / l_k)


def _out_body(oh_ref, canon_ref, orows_ref, v_ref, wo_ref, bo_ref, out_ref,
              *, n_head, t_rows, l_k):
    t = pl.program_id(1)
    ohd = oh_ref[0, :, pl.ds(t * t_rows, t_rows)] * canon_ref[0]
    meanv = jnp.sum(v_ref[0], axis=0, keepdims=True) / l_k
    selrow = lax.dot_general(ohd, jnp.ones((ohd.shape[0], 1), F32),
                             (((0,), (0,)), ((), ())),
                             precision=_PREC, preferred_element_type=F32)
    opre = lax.dot_general(ohd, orows_ref[0], (((0,), (0,)), ((), ())),
                           precision=_PREC, preferred_element_type=F32)
    opre = opre + (1.0 - selrow) * meanv
    y = lax.dot_general(opre, wo_ref[...], (((1,), (1,)), ((), ())),
                        precision=_PREC, preferred_element_type=F32) + bo_ref[...]
    out_ref[0] = y


def kernel(q, k, v, mask, Wq, bq, Wk, bk, Wv, bv, Wo, bo):
    del mask
    b_sz, l_q, d_model = q.shape
    l_k = k.shape[1]
    n_head = 12
    dk = d_model // n_head
    dv = Wv.shape[0]
    u = min(int(3 * np.ceil(np.log(l_q))), l_q)
    hu = n_head * u
    scale = 1.0 / sqrt(dk)
    t_rows = 256
    m_rows = 512

    cnts_t, _ = _sample_counts_t(l_q, l_k)

    qp_hi, qp_lo = _project(q.reshape(b_sz * l_q, d_model), Wq, bq, 512,
                            split=True)
    kp_hi, kp_lo = _project(k.reshape(b_sz * l_k, d_model), Wk, bk, 512,
                            split=True)
    vp = _project(v.reshape(b_sz * l_k, d_model), Wv, bv, 512)
    tr = lambda a, l: a.reshape(b_sz, l, n_head, dk).transpose(0, 2, 1, 3)
    q4h = tr(qp_hi, l_q)
    q4l = tr(qp_lo, l_q)
    k4h = tr(kp_hi, l_k)
    k4l = tr(kp_lo, l_k)
    v3 = vp.reshape(b_sz, l_k, dv)

    m = pl.pallas_call(
        functools.partial(_m_body, t_rows=m_rows, l_k=l_k),
        grid=(b_sz * n_head, l_q // m_rows),
        in_specs=[
            pl.BlockSpec((1, 1, m_rows, dk),
                         lambda bh, t, H=n_head: (bh // H, bh % H, t, 0)),
            pl.BlockSpec((1, 1, m_rows, dk),
                         lambda bh, t, H=n_head: (bh // H, bh % H, t, 0)),
            pl.BlockSpec((1, 1, l_k, dk),
                         lambda bh, t, H=n_head: (bh // H, bh % H, 0, 0)),
            pl.BlockSpec((1, 1, l_k, dk),
                         lambda bh, t, H=n_head: (bh // H, bh % H, 0, 0)),
            pl.BlockSpec((l_k, l_q), lambda bh, t: (0, 0)),
        ],
        out_specs=pl.BlockSpec((1, 1, l_q), lambda bh, t: (bh, 0, 0)),
        out_shape=jax.ShapeDtypeStruct((b_sz * n_head, 1, l_q), F32),
    )(q4h, q4l, k4h, k4l, cnts_t)

    bh_rows = b_sz * n_head
    oh, top_idx = pl.pallas_call(
        functools.partial(_topk_body, u=u, l_q=l_q, rows=bh_rows),
        grid=(1,),
        in_specs=[pl.BlockSpec((bh_rows, 1, l_q), lambda i: (0, 0, 0))],
        out_specs=[
            pl.BlockSpec((bh_rows, u, l_q), lambda i: (0, 0, 0)),
            pl.BlockSpec((bh_rows, u, 1), lambda i: (0, 0, 0)),
        ],
        out_shape=[
            jax.ShapeDtypeStruct((bh_rows, u, l_q), F32),
            jax.ShapeDtypeStruct((bh_rows, u, 1), jnp.int32),
        ],
    )(m)

    pall = pl.pallas_call(
        functools.partial(_softmax_rows_body, scale=scale),
        grid=(b_sz * n_head,),
        in_specs=[
            pl.BlockSpec((1, u, l_q), lambda bh: (bh, 0, 0)),
            pl.BlockSpec((1, 1, l_q, dk),
                         lambda bh, H=n_head: (bh // H, bh % H, 0, 0)),
            pl.BlockSpec((1, 1, l_q, dk),
                         lambda bh, H=n_head: (bh // H, bh % H, 0, 0)),
            pl.BlockSpec((1, 1, l_k, dk),
                         lambda bh, H=n_head: (bh // H, bh % H, 0, 0)),
            pl.BlockSpec((1, 1, l_k, dk),
                         lambda bh, H=n_head: (bh // H, bh % H, 0, 0)),
        ],
        out_specs=pl.BlockSpec((1, u, l_k), lambda bh: (bh, 0, 0)),
        out_shape=jax.ShapeDtypeStruct((b_sz * n_head, u, l_k), F32),
    )(oh, q4h, q4l, k4h, k4l)

    ohb3 = oh.reshape(b_sz, hu, l_q)
    pall3 = pall.reshape(b_sz, hu, l_k)

    pcomb, canon, orows = pl.pallas_call(
        functools.partial(_merge_body, n_head=n_head, u=u, l_k=l_k),
        grid=(b_sz,),
        in_specs=[
            pl.BlockSpec((1, hu, l_q), lambda b: (b, 0, 0)),
            pl.BlockSpec((1, hu, l_k), lambda b: (b, 0, 0)),
            pl.BlockSpec((1, l_k, dv), lambda b: (b, 0, 0)),
        ],
        out_specs=[
            pl.BlockSpec((1, hu, l_k), lambda b: (b, 0, 0)),
            pl.BlockSpec((1, hu, 1), lambda b: (b, 0, 0)),
            pl.BlockSpec((1, hu, dv), lambda b: (b, 0, 0)),
        ],
        out_shape=[
            jax.ShapeDtypeStruct((b_sz, hu, l_k), F32),
            jax.ShapeDtypeStruct((b_sz, hu, 1), F32),
            jax.ShapeDtypeStruct((b_sz, hu, dv), F32),
        ],
    )(ohb3, pall3, v3)

    attn = pl.pallas_call(
        functools.partial(_attn_body, n_head=n_head, t_rows=t_rows, l_k=l_k),
        grid=(b_sz, l_q // t_rows),
        in_specs=[
            pl.BlockSpec((1, hu, l_q), lambda b, t: (b, 0, 0)),
            pl.BlockSpec((1, hu, l_k), lambda b, t: (b, 0, 0)),
            pl.BlockSpec((1, hu, 1), lambda b, t: (b, 0, 0)),
        ],
        out_specs=pl.BlockSpec((1, t_rows, l_k), lambda b, t: (b, t, 0)),
        out_shape=jax.ShapeDtypeStruct((b_sz, l_q, l_k), F32),
    )(ohb3, pcomb, canon)

    out = pl.pallas_call(
        functools.partial(_out_body, n_head=n_head, t_rows=t_rows, l_k=l_k),
        grid=(b_sz, l_q // t_rows),
        in_specs=[
            pl.BlockSpec((1, hu, l_q), lambda b, t: (b, 0, 0)),
            pl.BlockSpec((1, hu, 1), lambda b, t: (b, 0, 0)),
            pl.BlockSpec((1, hu, dv), lambda b, t: (b, 0, 0)),
            pl.BlockSpec((1, l_k, dv), lambda b, t: (b, 0, 0)),
            pl.BlockSpec((d_model, dv), lambda b, t: (0, 0)),
            pl.BlockSpec((1, d_model), lambda b, t: (0, 0)),
        ],
        out_specs=pl.BlockSpec((1, t_rows, d_model), lambda b, t: (b, t, 0)),
        out_shape=jax.ShapeDtypeStruct((b_sz, l_q, d_model), F32),
    )(ohb3, canon, orows, v3, Wo, bo.reshape(1, d_model))

    return (out, attn)


# speedup vs baseline: 1.4360x; 1.0671x over previous
import functools
from math import sqrt

import numpy as np
import jax
import jax.numpy as jnp
from jax import lax
from jax.experimental import pallas as pl
from jax.experimental.pallas import tpu as pltpu

F32 = jnp.float32
_PREC = lax.Precision.HIGHEST
BF16 = jnp.bfloat16

_CONSTS = {}


def _threefry2x32(k0, k1, c0, c1):
    k0, k1 = np.uint32(k0), np.uint32(k1)
    x0 = (c0 + k0).astype(np.uint32)
    x1 = (c1 + k1).astype(np.uint32)
    ks = [k0, k1, np.uint32(np.uint32(k0) ^ np.uint32(k1) ^ np.uint32(0x1BD11BDA))]
    rots = [[13, 15, 26, 6], [17, 29, 16, 24]]
    for g in range(5):
        for r in rots[g % 2]:
            x0 = (x0 + x1).astype(np.uint32)
            x1 = ((x1 << np.uint32(r)) | (x1 >> np.uint32(32 - r))).astype(np.uint32) ^ x0
        x0 = (x0 + ks[(g + 1) % 3]).astype(np.uint32)
        x1 = (x1 + ks[(g + 2) % 3] + np.uint32(g + 1)).astype(np.uint32)
    return x0, x1


def _np_randint(shape, span):
    o0, o1 = _threefry2x32(np.uint32(0), np.uint32(42),
                           np.zeros(2, np.uint32), np.arange(2, dtype=np.uint32))
    n = int(np.prod(shape))

    def bits(sk0, sk1):
        c = np.arange(n, dtype=np.uint64)
        hi = (c >> np.uint64(32)).astype(np.uint32)
        lo = (c & np.uint64(0xFFFFFFFF)).astype(np.uint32)
        x0, x1 = _threefry2x32(sk0, sk1, hi, lo)
        return (x0 ^ x1).astype(np.uint32)

    u = bits(o0[0], o1[0])
    v = bits(o0[1], o1[1])
    be = np.uint32(span)
    bh = np.uint32((np.uint64(65536 % span) ** 2) % np.uint64(span))
    out = ((u % be) * bh + (v % be)) % be
    return out.astype(np.int32).reshape(shape)


def _sample_counts_t(l_q: int, l_k: int):
    ck = (l_q, l_k)
    if ck not in _CONSTS:
        u_part = min(int(3 * np.ceil(np.log(l_k))), l_k)
        idx_np = _np_randint((l_q, u_part), l_k)
        cnt = np.zeros((l_k, l_q), dtype=np.float32)
        rows = np.broadcast_to(np.arange(l_q)[:, None], idx_np.shape)
        np.add.at(cnt, (idx_np, rows), 1.0)
        neg = np.where(cnt > 0.0, 0.0, -1e30).astype(np.float32)
        _CONSTS[ck] = (jnp.asarray(cnt), jnp.asarray(neg))
    return _CONSTS[ck]


def _proj_body(x_ref, w_ref, b_ref, o_ref):
    o_ref[...] = (
        lax.dot_general(
            x_ref[...], w_ref[...], (((1,), (1,)), ((), ())),
            precision=_PREC, preferred_element_type=F32,
        )
        + b_ref[...]
    )


def _proj_split_body(x_ref, w_ref, b_ref, hi_ref, lo_ref):
    y = (
        lax.dot_general(
            x_ref[...], w_ref[...], (((1,), (1,)), ((), ())),
            precision=_PREC, preferred_element_type=F32,
        )
        + b_ref[...]
    )
    hi = y.astype(BF16)
    hi_ref[...] = hi
    lo_ref[...] = (y - hi.astype(F32)).astype(BF16)


def _project(x2d, w, b, tile, split=False):
    n, d_in = x2d.shape
    d_out = w.shape[0]
    in_specs = [
        pl.BlockSpec((tile, d_in), lambda i: (i, 0)),
        pl.BlockSpec((d_out, d_in), lambda i: (0, 0)),
        pl.BlockSpec((1, d_out), lambda i: (0, 0)),
    ]
    if not split:
        return pl.pallas_call(
            _proj_body,
            grid=(n // tile,),
            in_specs=in_specs,
            out_specs=pl.BlockSpec((tile, d_out), lambda i: (i, 0)),
            out_shape=jax.ShapeDtypeStruct((n, d_out), F32),
        )(x2d, w, b.reshape(1, d_out))
    return pl.pallas_call(
        _proj_split_body,
        grid=(n // tile,),
        in_specs=in_specs,
        out_specs=[
            pl.BlockSpec((tile, d_out), lambda i: (i, 0)),
            pl.BlockSpec((tile, d_out), lambda i: (i, 0)),
        ],
        out_shape=[
            jax.ShapeDtypeStruct((n, d_out), BF16),
            jax.ShapeDtypeStruct((n, d_out), BF16),
        ],
    )(x2d, w, b.reshape(1, d_out))


def _m_body(qh_ref, ql_ref, kh_ref, kl_ref, c_ref, m_ref, *, t_rows, l_k):
    t = pl.program_id(1)
    dims = (((1,), (1,)), ((), ()))
    qh = qh_ref[0, 0]
    ql = ql_ref[0, 0]
    kh = kh_ref[0, 0]
    kl = kl_ref[0, 0]
    st = (
        lax.dot_general(kh, qh, dims, preferred_element_type=F32)
        + lax.dot_general(kh, ql, dims, preferred_element_type=F32)
        + lax.dot_general(kl, qh, dims, preferred_element_type=F32)
    )
    c = c_ref[:, pl.ds(t * t_rows, t_rows)]
    smax = jnp.max(jnp.where(c > 0.0, st, -1e30), axis=0, keepdims=True)
    ssum = jnp.sum(st * c, axis=0, keepdims=True)
    m_ref[0, :, pl.ds(t * t_rows, t_rows)] = smax - ssum / l_k


def _topk_body(m_ref, oh_ref, idx_ref, *, u, l_q, rows):
    m = m_ref[:, 0, :]
    iota_q = lax.broadcasted_iota(jnp.int32, (rows, l_q), 1)
    iota_u = lax.broadcasted_iota(jnp.int32, (rows, u), 1)

    def body(j, carry):
        mcur, idx = carry
        mx = jnp.max(mcur, axis=1, keepdims=True)
        amax = jnp.min(jnp.where(mcur == mx, iota_q, l_q),
                       axis=1, keepdims=True)
        idx = jnp.where(iota_u == j, amax, idx)
        mcur = jnp.where(iota_q == amax, -1e30, mcur)
        return mcur, idx

    _, idx = lax.fori_loop(0, u, body,
                           (m, jnp.zeros((rows, u), jnp.int32)))
    iota3 = lax.broadcasted_iota(jnp.int32, (rows, u, l_q), 2)
    oh_ref[...] = (idx[:, :, None] == iota3).astype(F32)
    idx_ref[...] = idx[:, :, None]


def _softmax_rows_body(oh_ref, qh_ref, ql_ref, kh_ref, kl_ref, p_ref, *, scale):
    sel = (((1,), (0,)), ((), ()))
    dims = (((1,), (1,)), ((), ()))
    oh16 = oh_ref[0].astype(BF16)
    qred = (
        lax.dot_general(oh16, qh_ref[0, 0], sel, preferred_element_type=F32)
        + lax.dot_general(oh16, ql_ref[0, 0], sel, preferred_element_type=F32)
    )
    qrh = qred.astype(BF16)
    qrl = (qred - qrh.astype(F32)).astype(BF16)
    kh = kh_ref[0, 0]
    kl = kl_ref[0, 0]
    s = (
        lax.dot_general(qrh, kh, dims, preferred_element_type=F32)
        + lax.dot_general(qrh, kl, dims, preferred_element_type=F32)
        + lax.dot_general(qrl, kh, dims, preferred_element_type=F32)
    ) * scale
    p = jnp.exp(s - jnp.max(s, axis=1, keepdims=True))
    p_ref[0] = p / jnp.sum(p, axis=1, keepdims=True)


def _merge_body(oh_ref, p_ref, v_ref, pch_ref, pcl_ref, canon_ref, orows_ref,
                *, n_head, u, l_k):
    hu = n_head * u
    pall = p_ref[0]
    ohb = oh_ref[0]
    eq = lax.dot_general(ohb, ohb, (((1,), (1,)), ((), ())),
                         precision=_PREC, preferred_element_type=F32)
    cnt = jnp.sum(eq, axis=1, keepdims=True)
    ii = lax.broadcasted_iota(jnp.int32, (hu, hu), 0)
    jj = lax.broadcasted_iota(jnp.int32, (hu, hu), 1)
    prior = jnp.sum(eq * (jj < ii).astype(F32), axis=1, keepdims=True)
    canon = (prior == 0.0).astype(F32)
    base = (n_head - cnt) / (n_head * l_k)
    pc = base + lax.dot_general(eq, pall, (((1,), (0,)), ((), ())),
                                precision=_PREC, preferred_element_type=F32) / n_head
    pcc = pc * canon
    pch = pcc.astype(BF16)
    pch_ref[0] = pch
    pcl_ref[0] = (pcc - pch.astype(F32)).astype(BF16)
    canon_ref[0] = canon
    orows_ref[0] = lax.dot_general(pcc, v_ref[0], (((1,), (0,)), ((), ())),
                                   precision=_PREC, preferred_element_type=F32)


def _attn_body(oh_ref, pch_ref, pcl_ref, canon_ref, attn_ref,
               *, n_head, t_rows, l_k):
    t = pl.program_id(1)
    sca = (((0,), (0,)), ((), ()))
    ohd = (oh_ref[0, :, pl.ds(t * t_rows, t_rows)]
           * canon_ref[0]).astype(BF16)
    content = (
        lax.dot_general(ohd, pch_ref[0], sca, preferred_element_type=F32)
        + lax.dot_general(ohd, pcl_ref[0], sca, preferred_element_type=F32)
    )
    selrow = lax.dot_general(ohd, jnp.ones((ohd.shape[0], 1), BF16),
                             sca, preferred_element_type=F32)
    attn_ref[0] = content + (1.0 - selrow) * (1.0 / l_k)


def _out_body(oh_ref, canon_ref, orows_ref, v_ref, wo_ref, bo_ref, out_ref,
              *, n_head, t_rows, l_k):
    t = pl.program_id(1)
    ohd = oh_ref[0, :, pl.ds(t * t_rows, t_rows)] * canon_ref[0]
    meanv = jnp.sum(v_ref[0], axis=0, keepdims=True) / l_k
    selrow = lax.dot_general(ohd, jnp.ones((ohd.shape[0], 1), F32),
                             (((0,), (0,)), ((), ())),
                             precision=_PREC, preferred_element_type=F32)
    opre = lax.dot_general(ohd, orows_ref[0], (((0,), (0,)), ((), ())),
                           precision=_PREC, preferred_element_type=F32)
    opre = opre + (1.0 - selrow) * meanv
    y = lax.dot_general(opre, wo_ref[...], (((1,), (1,)), ((), ())),
                        precision=_PREC, preferred_element_type=F32) + bo_ref[...]
    out_ref[0] = y


def kernel(q, k, v, mask, Wq, bq, Wk, bk, Wv, bv, Wo, bo):
    del mask
    b_sz, l_q, d_model = q.shape
    l_k = k.shape[1]
    n_head = 12
    dk = d_model // n_head
    dv = Wv.shape[0]
    u = min(int(3 * np.ceil(np.log(l_q))), l_q)
    hu = n_head * u
    scale = 1.0 / sqrt(dk)
    t_rows = 256
    m_rows = 512

    cnts_t, _ = _sample_counts_t(l_q, l_k)

    qp_hi, qp_lo = _project(q.reshape(b_sz * l_q, d_model), Wq, bq, 512,
                            split=True)
    kp_hi, kp_lo = _project(k.reshape(b_sz * l_k, d_model), Wk, bk, 512,
                            split=True)
    vp = _project(v.reshape(b_sz * l_k, d_model), Wv, bv, 512)
    tr = lambda a, l: a.reshape(b_sz, l, n_head, dk).transpose(0, 2, 1, 3)
    q4h = tr(qp_hi, l_q)
    q4l = tr(qp_lo, l_q)
    k4h = tr(kp_hi, l_k)
    k4l = tr(kp_lo, l_k)
    v3 = vp.reshape(b_sz, l_k, dv)

    m = pl.pallas_call(
        functools.partial(_m_body, t_rows=m_rows, l_k=l_k),
        grid=(b_sz * n_head, l_q // m_rows),
        in_specs=[
            pl.BlockSpec((1, 1, m_rows, dk),
                         lambda bh, t, H=n_head: (bh // H, bh % H, t, 0)),
            pl.BlockSpec((1, 1, m_rows, dk),
                         lambda bh, t, H=n_head: (bh // H, bh % H, t, 0)),
            pl.BlockSpec((1, 1, l_k, dk),
                         lambda bh, t, H=n_head: (bh // H, bh % H, 0, 0)),
            pl.BlockSpec((1, 1, l_k, dk),
                         lambda bh, t, H=n_head: (bh // H, bh % H, 0, 0)),
            pl.BlockSpec((l_k, l_q), lambda bh, t: (0, 0)),
        ],
        out_specs=pl.BlockSpec((1, 1, l_q), lambda bh, t: (bh, 0, 0)),
        out_shape=jax.ShapeDtypeStruct((b_sz * n_head, 1, l_q), F32),
    )(q4h, q4l, k4h, k4l, cnts_t)

    bh_rows = b_sz * n_head
    oh, top_idx = pl.pallas_call(
        functools.partial(_topk_body, u=u, l_q=l_q, rows=bh_rows),
        grid=(1,),
        in_specs=[pl.BlockSpec((bh_rows, 1, l_q), lambda i: (0, 0, 0))],
        out_specs=[
            pl.BlockSpec((bh_rows, u, l_q), lambda i: (0, 0, 0)),
            pl.BlockSpec((bh_rows, u, 1), lambda i: (0, 0, 0)),
        ],
        out_shape=[
            jax.ShapeDtypeStruct((bh_rows, u, l_q), F32),
            jax.ShapeDtypeStruct((bh_rows, u, 1), jnp.int32),
        ],
    )(m)

    pall = pl.pallas_call(
        functools.partial(_softmax_rows_body, scale=scale),
        grid=(b_sz * n_head,),
        in_specs=[
            pl.BlockSpec((1, u, l_q), lambda bh: (bh, 0, 0)),
            pl.BlockSpec((1, 1, l_q, dk),
                         lambda bh, H=n_head: (bh // H, bh % H, 0, 0)),
            pl.BlockSpec((1, 1, l_q, dk),
                         lambda bh, H=n_head: (bh // H, bh % H, 0, 0)),
            pl.BlockSpec((1, 1, l_k, dk),
                         lambda bh, H=n_head: (bh // H, bh % H, 0, 0)),
            pl.BlockSpec((1, 1, l_k, dk),
                         lambda bh, H=n_head: (bh // H, bh % H, 0, 0)),
        ],
        out_specs=pl.BlockSpec((1, u, l_k), lambda bh: (bh, 0, 0)),
        out_shape=jax.ShapeDtypeStruct((b_sz * n_head, u, l_k), F32),
    )(oh, q4h, q4l, k4h, k4l)

    ohb3 = oh.reshape(b_sz, hu, l_q)
    pall3 = pall.reshape(b_sz, hu, l_k)

    pch, pcl, canon, orows = pl.pallas_call(
        functools.partial(_merge_body, n_head=n_head, u=u, l_k=l_k),
        grid=(b_sz,),
        in_specs=[
            pl.BlockSpec((1, hu, l_q), lambda b: (b, 0, 0)),
            pl.BlockSpec((1, hu, l_k), lambda b: (b, 0, 0)),
            pl.BlockSpec((1, l_k, dv), lambda b: (b, 0, 0)),
        ],
        out_specs=[
            pl.BlockSpec((1, hu, l_k), lambda b: (b, 0, 0)),
            pl.BlockSpec((1, hu, l_k), lambda b: (b, 0, 0)),
            pl.BlockSpec((1, hu, 1), lambda b: (b, 0, 0)),
            pl.BlockSpec((1, hu, dv), lambda b: (b, 0, 0)),
        ],
        out_shape=[
            jax.ShapeDtypeStruct((b_sz, hu, l_k), BF16),
            jax.ShapeDtypeStruct((b_sz, hu, l_k), BF16),
            jax.ShapeDtypeStruct((b_sz, hu, 1), F32),
            jax.ShapeDtypeStruct((b_sz, hu, dv), F32),
        ],
    )(ohb3, pall3, v3)

    attn = pl.pallas_call(
        functools.partial(_attn_body, n_head=n_head, t_rows=t_rows, l_k=l_k),
        grid=(b_sz, l_q // t_rows),
        in_specs=[
            pl.BlockSpec((1, hu, l_q), lambda b, t: (b, 0, 0)),
            pl.BlockSpec((1, hu, l_k), lambda b, t: (b, 0, 0)),
            pl.BlockSpec((1, hu, l_k), lambda b, t: (b, 0, 0)),
            pl.BlockSpec((1, hu, 1), lambda b, t: (b, 0, 0)),
        ],
        out_specs=pl.BlockSpec((1, t_rows, l_k), lambda b, t: (b, t, 0)),
        out_shape=jax.ShapeDtypeStruct((b_sz, l_q, l_k), F32),
    )(ohb3, pch, pcl, canon)

    out = pl.pallas_call(
        functools.partial(_out_body, n_head=n_head, t_rows=t_rows, l_k=l_k),
        grid=(b_sz, l_q // t_rows),
        in_specs=[
            pl.BlockSpec((1, hu, l_q), lambda b, t: (b, 0, 0)),
            pl.BlockSpec((1, hu, 1), lambda b, t: (b, 0, 0)),
            pl.BlockSpec((1, hu, dv), lambda b, t: (b, 0, 0)),
            pl.BlockSpec((1, l_k, dv), lambda b, t: (b, 0, 0)),
            pl.BlockSpec((d_model, dv), lambda b, t: (0, 0)),
            pl.BlockSpec((1, d_model), lambda b, t: (0, 0)),
        ],
        out_specs=pl.BlockSpec((1, t_rows, d_model), lambda b, t: (b, t, 0)),
        out_shape=jax.ShapeDtypeStruct((b_sz, l_q, d_model), F32),
    )(ohb3, canon, orows, v3, Wo, bo.reshape(1, d_model))

    return (out, attn)


# speedup vs baseline: 1.4655x; 1.0205x over previous
import functools
from math import sqrt

import numpy as np
import jax
import jax.numpy as jnp
from jax import lax
from jax.experimental import pallas as pl
from jax.experimental.pallas import tpu as pltpu

F32 = jnp.float32
_PREC = lax.Precision.HIGHEST
BF16 = jnp.bfloat16

_CONSTS = {}


def _threefry2x32(k0, k1, c0, c1):
    k0, k1 = np.uint32(k0), np.uint32(k1)
    x0 = (c0 + k0).astype(np.uint32)
    x1 = (c1 + k1).astype(np.uint32)
    ks = [k0, k1, np.uint32(np.uint32(k0) ^ np.uint32(k1) ^ np.uint32(0x1BD11BDA))]
    rots = [[13, 15, 26, 6], [17, 29, 16, 24]]
    for g in range(5):
        for r in rots[g % 2]:
            x0 = (x0 + x1).astype(np.uint32)
            x1 = ((x1 << np.uint32(r)) | (x1 >> np.uint32(32 - r))).astype(np.uint32) ^ x0
        x0 = (x0 + ks[(g + 1) % 3]).astype(np.uint32)
        x1 = (x1 + ks[(g + 2) % 3] + np.uint32(g + 1)).astype(np.uint32)
    return x0, x1


def _np_randint(shape, span):
    o0, o1 = _threefry2x32(np.uint32(0), np.uint32(42),
                           np.zeros(2, np.uint32), np.arange(2, dtype=np.uint32))
    n = int(np.prod(shape))

    def bits(sk0, sk1):
        c = np.arange(n, dtype=np.uint64)
        hi = (c >> np.uint64(32)).astype(np.uint32)
        lo = (c & np.uint64(0xFFFFFFFF)).astype(np.uint32)
        x0, x1 = _threefry2x32(sk0, sk1, hi, lo)
        return (x0 ^ x1).astype(np.uint32)

    u = bits(o0[0], o1[0])
    v = bits(o0[1], o1[1])
    be = np.uint32(span)
    bh = np.uint32((np.uint64(65536 % span) ** 2) % np.uint64(span))
    out = ((u % be) * bh + (v % be)) % be
    return out.astype(np.int32).reshape(shape)


def _sample_counts_t(l_q: int, l_k: int):
    ck = (l_q, l_k)
    if ck not in _CONSTS:
        u_part = min(int(3 * np.ceil(np.log(l_k))), l_k)
        idx_np = _np_randint((l_q, u_part), l_k)
        cnt = np.zeros((l_k, l_q), dtype=np.float32)
        rows = np.broadcast_to(np.arange(l_q)[:, None], idx_np.shape)
        np.add.at(cnt, (idx_np, rows), 1.0)
        neg = np.where(cnt > 0.0, 0.0, -1e30).astype(np.float32)
        _CONSTS[ck] = (jnp.asarray(cnt), jnp.asarray(neg))
    return _CONSTS[ck]


def _proj_body(x_ref, w_ref, b_ref, o_ref):
    o_ref[...] = (
        lax.dot_general(
            x_ref[...], w_ref[...], (((1,), (1,)), ((), ())),
            precision=_PREC, preferred_element_type=F32,
        )
        + b_ref[...]
    )


def _proj_split_body(x_ref, w_ref, b_ref, hi_ref, lo_ref):
    y = (
        lax.dot_general(
            x_ref[...], w_ref[...], (((1,), (1,)), ((), ())),
            precision=_PREC, preferred_element_type=F32,
        )
        + b_ref[...]
    )
    hi = y.astype(BF16)
    hi_ref[...] = hi
    lo_ref[...] = (y - hi.astype(F32)).astype(BF16)


def _project(x2d, w, b, tile, split=False):
    n, d_in = x2d.shape
    d_out = w.shape[0]
    in_specs = [
        pl.BlockSpec((tile, d_in), lambda i: (i, 0)),
        pl.BlockSpec((d_out, d_in), lambda i: (0, 0)),
        pl.BlockSpec((1, d_out), lambda i: (0, 0)),
    ]
    if not split:
        return pl.pallas_call(
            _proj_body,
            grid=(n // tile,),
            in_specs=in_specs,
            out_specs=pl.BlockSpec((tile, d_out), lambda i: (i, 0)),
            out_shape=jax.ShapeDtypeStruct((n, d_out), F32),
        )(x2d, w, b.reshape(1, d_out))
    return pl.pallas_call(
        _proj_split_body,
        grid=(n // tile,),
        in_specs=in_specs,
        out_specs=[
            pl.BlockSpec((tile, d_out), lambda i: (i, 0)),
            pl.BlockSpec((tile, d_out), lambda i: (i, 0)),
        ],
        out_shape=[
            jax.ShapeDtypeStruct((n, d_out), BF16),
            jax.ShapeDtypeStruct((n, d_out), BF16),
        ],
    )(x2d, w, b.reshape(1, d_out))


def _m_body(qh_ref, ql_ref, kh_ref, kl_ref, c_ref, m_ref, *, t_rows, l_k):
    t = pl.program_id(1)
    dims = (((1,), (1,)), ((), ()))
    qh = qh_ref[0, 0]
    ql = ql_ref[0, 0]
    kh = kh_ref[0, 0]
    kl = kl_ref[0, 0]
    st = (
        lax.dot_general(kh, qh, dims, preferred_element_type=F32)
        + lax.dot_general(kh, ql, dims, preferred_element_type=F32)
        + lax.dot_general(kl, qh, dims, preferred_element_type=F32)
    )
    c = c_ref[:, pl.ds(t * t_rows, t_rows)]
    smax = jnp.max(jnp.where(c > 0.0, st, -1e30), axis=0, keepdims=True)
    ssum = jnp.sum(st * c, axis=0, keepdims=True)
    m_ref[0, :, pl.ds(t * t_rows, t_rows)] = smax - ssum / l_k


def _topk_body(m_ref, oh_ref, idx_ref, *, u, l_q, rows):
    m = m_ref[:, 0, :]
    iota_q = lax.broadcasted_iota(jnp.int32, (rows, l_q), 1)
    iota_u = lax.broadcasted_iota(jnp.int32, (rows, u), 1)

    def body(j, carry):
        mcur, idx = carry
        mx = jnp.max(mcur, axis=1, keepdims=True)
        amax = jnp.min(jnp.where(mcur == mx, iota_q, l_q),
                       axis=1, keepdims=True)
        idx = jnp.where(iota_u == j, amax, idx)
        mcur = jnp.where(iota_q == amax, -1e30, mcur)
        return mcur, idx

    _, idx = lax.fori_loop(0, u, body,
                           (m, jnp.zeros((rows, u), jnp.int32)))
    iota3 = lax.broadcasted_iota(jnp.int32, (rows, u, l_q), 2)
    oh_ref[...] = (idx[:, :, None] == iota3).astype(F32)
    idx_ref[...] = idx[:, :, None]


def _softmax_rows_body(oh_ref, qh_ref, ql_ref, kh_ref, kl_ref, p_ref, *, scale):
    sel = (((1,), (0,)), ((), ()))
    dims = (((1,), (1,)), ((), ()))
    oh16 = oh_ref[0].astype(BF16)
    qred = (
        lax.dot_general(oh16, qh_ref[0, 0], sel, preferred_element_type=F32)
        + lax.dot_general(oh16, ql_ref[0, 0], sel, preferred_element_type=F32)
    )
    qrh = qred.astype(BF16)
    qrl = (qred - qrh.astype(F32)).astype(BF16)
    kh = kh_ref[0, 0]
    kl = kl_ref[0, 0]
    s = (
        lax.dot_general(qrh, kh, dims, preferred_element_type=F32)
        + lax.dot_general(qrh, kl, dims, preferred_element_type=F32)
        + lax.dot_general(qrl, kh, dims, preferred_element_type=F32)
    ) * scale
    p = jnp.exp(s - jnp.max(s, axis=1, keepdims=True))
    p_ref[0] = p / jnp.sum(p, axis=1, keepdims=True)


def _merge_body(oh_ref, p_ref, v_ref, pch_ref, pcl_ref, canon_ref, orows_ref,
                *, n_head, u, l_k):
    hu = n_head * u
    pall = p_ref[0]
    ohb = oh_ref[0]
    eq = lax.dot_general(ohb, ohb, (((1,), (1,)), ((), ())),
                         precision=_PREC, preferred_element_type=F32)
    cnt = jnp.sum(eq, axis=1, keepdims=True)
    ii = lax.broadcasted_iota(jnp.int32, (hu, hu), 0)
    jj = lax.broadcasted_iota(jnp.int32, (hu, hu), 1)
    prior = jnp.sum(eq * (jj < ii).astype(F32), axis=1, keepdims=True)
    canon = (prior == 0.0).astype(F32)
    base = (n_head - cnt) / (n_head * l_k)
    pc = base + lax.dot_general(eq, pall, (((1,), (0,)), ((), ())),
                                precision=_PREC, preferred_element_type=F32) / n_head
    pcc = pc * canon
    pch = pcc.astype(BF16)
    pch_ref[0] = pch
    pcl_ref[0] = (pcc - pch.astype(F32)).astype(BF16)
    canon_ref[0] = canon
    orows_ref[0] = lax.dot_general(pcc, v_ref[0], (((1,), (0,)), ((), ())),
                                   precision=_PREC, preferred_element_type=F32)


def _attn_body(oh_ref, pch_ref, pcl_ref, canon_ref, attn_ref,
               *, n_head, t_rows, l_k):
    t = pl.program_id(1)
    sca = (((0,), (0,)), ((), ()))
    ohd = (oh_ref[0, :, pl.ds(t * t_rows, t_rows)]
           * canon_ref[0]).astype(BF16)
    content = (
        lax.dot_general(ohd, pch_ref[0], sca, preferred_element_type=F32)
        + lax.dot_general(ohd, pcl_ref[0], sca, preferred_element_type=F32)
    )
    selrow = lax.dot_general(ohd, jnp.ones((ohd.shape[0], 1), BF16),
                             sca, preferred_element_type=F32)
    attn_ref[0] = content + (1.0 - selrow) * (1.0 / l_k)


def _out_body(oh_ref, canon_ref, orows_ref, v_ref, wo_ref, bo_ref, out_ref,
              *, n_head, t_rows, l_k):
    t = pl.program_id(1)
    ohd = oh_ref[0, :, pl.ds(t * t_rows, t_rows)] * canon_ref[0]
    meanv = jnp.sum(v_ref[0], axis=0, keepdims=True) / l_k
    selrow = lax.dot_general(ohd, jnp.ones((ohd.shape[0], 1), F32),
                             (((0,), (0,)), ((), ())),
                             precision=_PREC, preferred_element_type=F32)
    opre = lax.dot_general(ohd, orows_ref[0], (((0,), (0,)), ((), ())),
                           precision=_PREC, preferred_element_type=F32)
    opre = opre + (1.0 - selrow) * meanv
    y = lax.dot_general(opre, wo_ref[...], (((1,), (1,)), ((), ())),
                        precision=_PREC, preferred_element_type=F32) + bo_ref[...]
    out_ref[0] = y


def kernel(q, k, v, mask, Wq, bq, Wk, bk, Wv, bv, Wo, bo):
    del mask
    b_sz, l_q, d_model = q.shape
    l_k = k.shape[1]
    n_head = 12
    dk = d_model // n_head
    dv = Wv.shape[0]
    u = min(int(3 * np.ceil(np.log(l_q))), l_q)
    hu = n_head * u
    scale = 1.0 / sqrt(dk)
    t_rows = 256
    m_rows = 1024

    cnts_t, _ = _sample_counts_t(l_q, l_k)

    qp_hi, qp_lo = _project(q.reshape(b_sz * l_q, d_model), Wq, bq, 512,
                            split=True)
    kp_hi, kp_lo = _project(k.reshape(b_sz * l_k, d_model), Wk, bk, 512,
                            split=True)
    vp = _project(v.reshape(b_sz * l_k, d_model), Wv, bv, 512)
    tr = lambda a, l: a.reshape(b_sz, l, n_head, dk).transpose(0, 2, 1, 3)
    q4h = tr(qp_hi, l_q)
    q4l = tr(qp_lo, l_q)
    k4h = tr(kp_hi, l_k)
    k4l = tr(kp_lo, l_k)
    v3 = vp.reshape(b_sz, l_k, dv)

    m = pl.pallas_call(
        functools.partial(_m_body, t_rows=m_rows, l_k=l_k),
        grid=(b_sz * n_head, l_q // m_rows),
        in_specs=[
            pl.BlockSpec((1, 1, m_rows, dk),
                         lambda bh, t, H=n_head: (bh // H, bh % H, t, 0)),
            pl.BlockSpec((1, 1, m_rows, dk),
                         lambda bh, t, H=n_head: (bh // H, bh % H, t, 0)),
            pl.BlockSpec((1, 1, l_k, dk),
                         lambda bh, t, H=n_head: (bh // H, bh % H, 0, 0)),
            pl.BlockSpec((1, 1, l_k, dk),
                         lambda bh, t, H=n_head: (bh // H, bh % H, 0, 0)),
            pl.BlockSpec((l_k, l_q), lambda bh, t: (0, 0)),
        ],
        out_specs=pl.BlockSpec((1, 1, l_q), lambda bh, t: (bh, 0, 0)),
        out_shape=jax.ShapeDtypeStruct((b_sz * n_head, 1, l_q), F32),
    )(q4h, q4l, k4h, k4l, cnts_t)

    bh_rows = b_sz * n_head
    oh, top_idx = pl.pallas_call(
        functools.partial(_topk_body, u=u, l_q=l_q, rows=bh_rows),
        grid=(1,),
        in_specs=[pl.BlockSpec((bh_rows, 1, l_q), lambda i: (0, 0, 0))],
        out_specs=[
            pl.BlockSpec((bh_rows, u, l_q), lambda i: (0, 0, 0)),
            pl.BlockSpec((bh_rows, u, 1), lambda i: (0, 0, 0)),
        ],
        out_shape=[
            jax.ShapeDtypeStruct((bh_rows, u, l_q), F32),
            jax.ShapeDtypeStruct((bh_rows, u, 1), jnp.int32),
        ],
    )(m)

    pall = pl.pallas_call(
        functools.partial(_softmax_rows_body, scale=scale),
        grid=(b_sz * n_head,),
        in_specs=[
            pl.BlockSpec((1, u, l_q), lambda bh: (bh, 0, 0)),
            pl.BlockSpec((1, 1, l_q, dk),
                         lambda bh, H=n_head: (bh // H, bh % H, 0, 0)),
            pl.BlockSpec((1, 1, l_q, dk),
                         lambda bh, H=n_head: (bh // H, bh % H, 0, 0)),
            pl.BlockSpec((1, 1, l_k, dk),
                         lambda bh, H=n_head: (bh // H, bh % H, 0, 0)),
            pl.BlockSpec((1, 1, l_k, dk),
                         lambda bh, H=n_head: (bh // H, bh % H, 0, 0)),
        ],
        out_specs=pl.BlockSpec((1, u, l_k), lambda bh: (bh, 0, 0)),
        out_shape=jax.ShapeDtypeStruct((b_sz * n_head, u, l_k), F32),
    )(oh, q4h, q4l, k4h, k4l)

    ohb3 = oh.reshape(b_sz, hu, l_q)
    pall3 = pall.reshape(b_sz, hu, l_k)

    pch, pcl, canon, orows = pl.pallas_call(
        functools.partial(_merge_body, n_head=n_head, u=u, l_k=l_k),
        grid=(b_sz,),
        in_specs=[
            pl.BlockSpec((1, hu, l_q), lambda b: (b, 0, 0)),
            pl.BlockSpec((1, hu, l_k), lambda b: (b, 0, 0)),
            pl.BlockSpec((1, l_k, dv), lambda b: (b, 0, 0)),
        ],
        out_specs=[
            pl.BlockSpec((1, hu, l_k), lambda b: (b, 0, 0)),
            pl.BlockSpec((1, hu, l_k), lambda b: (b, 0, 0)),
            pl.BlockSpec((1, hu, 1), lambda b: (b, 0, 0)),
            pl.BlockSpec((1, hu, dv), lambda b: (b, 0, 0)),
        ],
        out_shape=[
            jax.ShapeDtypeStruct((b_sz, hu, l_k), BF16),
            jax.ShapeDtypeStruct((b_sz, hu, l_k), BF16),
            jax.ShapeDtypeStruct((b_sz, hu, 1), F32),
            jax.ShapeDtypeStruct((b_sz, hu, dv), F32),
        ],
    )(ohb3, pall3, v3)

    attn = pl.pallas_call(
        functools.partial(_attn_body, n_head=n_head, t_rows=t_rows, l_k=l_k),
        grid=(b_sz, l_q // t_rows),
        in_specs=[
            pl.BlockSpec((1, hu, l_q), lambda b, t: (b, 0, 0)),
            pl.BlockSpec((1, hu, l_k), lambda b, t: (b, 0, 0)),
            pl.BlockSpec((1, hu, l_k), lambda b, t: (b, 0, 0)),
            pl.BlockSpec((1, hu, 1), lambda b, t: (b, 0, 0)),
        ],
        out_specs=pl.BlockSpec((1, t_rows, l_k), lambda b, t: (b, t, 0)),
        out_shape=jax.ShapeDtypeStruct((b_sz, l_q, l_k), F32),
    )(ohb3, pch, pcl, canon)

    out = pl.pallas_call(
        functools.partial(_out_body, n_head=n_head, t_rows=t_rows, l_k=l_k),
        grid=(b_sz, l_q // t_rows),
        in_specs=[
            pl.BlockSpec((1, hu, l_q), lambda b, t: (b, 0, 0)),
            pl.BlockSpec((1, hu, 1), lambda b, t: (b, 0, 0)),
            pl.BlockSpec((1, hu, dv), lambda b, t: (b, 0, 0)),
            pl.BlockSpec((1, l_k, dv), lambda b, t: (b, 0, 0)),
            pl.BlockSpec((d_model, dv), lambda b, t: (0, 0)),
            pl.BlockSpec((1, d_model), lambda b, t: (0, 0)),
        ],
        out_specs=pl.BlockSpec((1, t_rows, d_model), lambda b, t: (b, t, 0)),
        out_shape=jax.ShapeDtypeStruct((b_sz, l_q, d_model), F32),
    )(ohb3, canon, orows, v3, Wo, bo.reshape(1, d_model))

    return (out, attn)
